# Initial kernel scaffold; baseline (speedup 1.0000x reference)
import functools
import jax
import jax.numpy as jnp
import numpy as np
from jax import lax
from jax.experimental import pallas as pl
from jax.experimental.pallas import tpu as pltpu
from jax.experimental.pallas import tpu_sc as plsc

NN = 50000
TT = 518
S1 = 114
S2 = 83
KW = 32
E0 = 800000
EPAD = 819200
EROWS = EPAD // 128
NPAD = 51200
NC, NS = 2, 16
BN = 2000
GRID_N = NN // BN

_mesh = plsc.VectorSubcoreMesh(
    core_axis_name="c", subcore_axis_name="s", num_cores=NC, num_subcores=NS)


def _k1_body(x_ref, a1_ref, a2_ref, w1t_ref, sb_ref, ha_ref, hb_ref):
    x = x_ref[...]
    t1 = jnp.maximum(
        jnp.dot(x, a1_ref[...], preferred_element_type=jnp.float32)
        + sb_ref[0, 0], 0.0)
    t2 = jnp.maximum(
        jnp.dot(t1, a2_ref[...], preferred_element_type=jnp.float32)
        + sb_ref[0, 1], 0.0)
    h = jnp.dot(t2, w1t_ref[...], preferred_element_type=jnp.float32)
    ha_ref[...] = h[:, :32]
    hb_ref[...] = h[:, 32:]


def _dense_stage(nf, a1f, a2, w1t, sb):
    return pl.pallas_call(
        _k1_body,
        grid=(GRID_N,),
        in_specs=[
            pl.BlockSpec((BN, TT), lambda i: (i, 0)),
            pl.BlockSpec((TT, S1), lambda i: (0, 0)),
            pl.BlockSpec((S1, S2), lambda i: (0, 0)),
            pl.BlockSpec((S2, 64), lambda i: (0, 0)),
            pl.BlockSpec((1, 2), lambda i: (0, 0)),
        ],
        out_specs=[
            pl.BlockSpec((BN, 32), lambda i: (i, 0)),
            pl.BlockSpec((BN, 32), lambda i: (i, 0)),
        ],
        out_shape=[
            jax.ShapeDtypeStruct((NN, 32), jnp.float32),
            jax.ShapeDtypeStruct((NN, 32), jnp.float32),
        ],
    )(nf, a1f, a2, w1t, sb)


def _dis_body(dp_ref, dis_ref):
    p = dp_ref[...]
    dis_ref[...] = lax.rsqrt(1.0 + p[0] + p[1])


def _dis_stage(degp):
    return pl.pallas_call(
        _dis_body,
        grid=(8,),
        in_specs=[pl.BlockSpec((2, NPAD // 8), lambda i: (0, i))],
        out_specs=pl.BlockSpec((NPAD // 8,), lambda i: (i,)),
        out_shape=jax.ShapeDtypeStruct((NPAD,), jnp.float32),
    )(degp)


def _zero_1d(ref, n):
    def body(i, _):
        ref[pl.ds(i * 16, 16)] = jnp.zeros((16,), jnp.float32)
        return 0
    lax.fori_loop(0, n // 16, body, 0)


def _k2_body(dst2, ew2, degp, idx_v, val_v, stripe_v, acc_sh, sem):
    cid = lax.axis_index("c")
    sid = lax.axis_index("s")
    wid = sid * NC + cid
    nbase = pl.multiple_of(sid * (NPAD // NS), 8)

    _zero_1d(stripe_v, NPAD // NS)
    pltpu.sync_copy(stripe_v, acc_sh.at[pl.ds(nbase, NPAD // NS)])
    plsc.subcore_barrier()

    rows_per_tile = EROWS // (NC * NS)
    rowbase = wid * rows_per_tile

    def chunk(ch, _):
        r0 = rowbase + ch * 8
        pltpu.sync_copy(dst2.at[pl.ds(r0, 8)], idx_v)
        pltpu.sync_copy(ew2.at[pl.ds(r0, 8)], val_v)
        pltpu.sync_copy(val_v, acc_sh.at[idx_v], add=True)
        return 0
    lax.fori_loop(0, rows_per_tile // 8, chunk, 0)

    plsc.subcore_barrier()
    pltpu.sync_copy(acc_sh.at[pl.ds(nbase, NPAD // NS)], stripe_v)
    pltpu.sync_copy(stripe_v, degp.at[cid, pl.ds(nbase, NPAD // NS)])


def _deg_stage(dst2, ew2):
    f = pl.kernel(
        _k2_body,
        out_type=jax.ShapeDtypeStruct((NC, NPAD), jnp.float32),
        mesh=_mesh,
        scratch_types=[
            pltpu.VMEM((8, 128), jnp.int32),
            pltpu.VMEM((8, 128), jnp.float32),
            pltpu.VMEM((NPAD // NS,), jnp.float32),
            pltpu.VMEM_SHARED((NPAD,), jnp.float32),
            pltpu.SemaphoreType.DMA,
        ],
    )
    return f(dst2, ew2)


def _k3_body(src2, dst2, ew2, dis, norm2, cp, sidx, didx, ewv, dsv, ddv, nv,
             stripe_v, acc_sh, sem):
    cid = lax.axis_index("c")
    sid = lax.axis_index("s")
    wid = sid * NC + cid
    nbase = pl.multiple_of(sid * (NPAD // NS), 8)

    _zero_1d(stripe_v, NPAD // NS)
    pltpu.sync_copy(stripe_v, acc_sh.at[pl.ds(nbase, NPAD // NS)])
    plsc.subcore_barrier()

    rows_per_tile = EROWS // (NC * NS)
    rowbase = wid * rows_per_tile

    def chunk(ch, _):
        r0 = rowbase + ch * 8
        pltpu.sync_copy(src2.at[pl.ds(r0, 8)], sidx)
        pltpu.sync_copy(dst2.at[pl.ds(r0, 8)], didx)
        pltpu.sync_copy(ew2.at[pl.ds(r0, 8)], ewv)
        pltpu.sync_copy(dis.at[sidx], dsv)
        pltpu.sync_copy(dis.at[didx], ddv)
        for j in range(8):
            for i in range(8):
                sl = pl.ds(i * 16, 16)
                nv[j, sl] = dsv[j, sl] * ewv[j, sl] * ddv[j, sl]
        pltpu.sync_copy(nv, norm2.at[pl.ds(r0, 8)])
        pltpu.sync_copy(nv, acc_sh.at[sidx], add=True)
        return 0
    lax.fori_loop(0, rows_per_tile // 8, chunk, 0)

    plsc.subcore_barrier()
    pltpu.sync_copy(acc_sh.at[pl.ds(nbase, NPAD // NS)], stripe_v)
    pltpu.sync_copy(stripe_v, cp.at[cid, pl.ds(nbase, NPAD // NS)])


def _norm_stage(src2, dst2, ew2, dis):
    f = pl.kernel(
        _k3_body,
        out_type=(
            jax.ShapeDtypeStruct((EROWS, 128), jnp.float32),
            jax.ShapeDtypeStruct((NC, NPAD), jnp.float32),
        ),
        mesh=_mesh,
        scratch_types=[
            pltpu.VMEM((8, 128), jnp.int32),
            pltpu.VMEM((8, 128), jnp.int32),
            pltpu.VMEM((8, 128), jnp.float32),
            pltpu.VMEM((8, 128), jnp.float32),
            pltpu.VMEM((8, 128), jnp.float32),
            pltpu.VMEM((8, 128), jnp.float32),
            pltpu.VMEM((NPAD // NS,), jnp.float32),
            pltpu.VMEM_SHARED((NPAD,), jnp.float32),
            pltpu.SemaphoreType.DMA,
        ],
    )
    return f(src2, dst2, ew2, dis)


def _k4_body(src2, dst2, norm2, ha, hb, agg, sidx, didx, nv, rows_v, zb,
             acc_sh, sem):
    cid = lax.axis_index("c")
    sid = lax.axis_index("s")
    nbase = pl.multiple_of(sid * (NPAD // NS), 8)

    def zb_zero(i, _):
        zb[i, pl.ds(0, 16)] = jnp.zeros((16,), jnp.float32)
        zb[i, pl.ds(16, 16)] = jnp.zeros((16,), jnp.float32)
        return 0
    lax.fori_loop(0, 1024, zb_zero, 0)
    for t in range(3):
        pltpu.sync_copy(zb, acc_sh.at[pl.ds(nbase + t * 1024, 1024)])
    pltpu.sync_copy(zb.at[pl.ds(0, 128)],
                    acc_sh.at[pl.ds(nbase + 3072, 128)])
    plsc.subcore_barrier()

    rows_per_tile = EROWS // NS
    rowbase = sid * rows_per_tile

    def run(table):
        def chunk(ch, _):
            r0 = rowbase + ch * 8
            pltpu.sync_copy(src2.at[pl.ds(r0, 8)], sidx)
            pltpu.sync_copy(dst2.at[pl.ds(r0, 8)], didx)
            pltpu.sync_copy(norm2.at[pl.ds(r0, 8)], nv)
            pltpu.sync_copy(table.at[sidx], rows_v)

            def scale(e, _):
                j = e // 128
                k = e - j * 128
                n = nv[j, k]
                rows_v[j, k, pl.ds(0, 16)] = rows_v[j, k, pl.ds(0, 16)] * n
                rows_v[j, k, pl.ds(16, 16)] = rows_v[j, k, pl.ds(16, 16)] * n
                return 0
            lax.fori_loop(0, 1024, scale, 0)

            pltpu.sync_copy(rows_v, acc_sh.at[didx], add=True)
            return 0
        lax.fori_loop(0, rows_per_tile // 8, chunk, 0)

    @pl.when(cid == 0)
    def _():
        run(ha)

    @pl.when(cid == 1)
    def _():
        run(hb)

    plsc.subcore_barrier()
    for t in range(3):
        pltpu.sync_copy(acc_sh.at[pl.ds(nbase + t * 1024, 1024)], zb)
        pltpu.sync_copy(zb, agg.at[cid, pl.ds(nbase + t * 1024, 1024)])
    pltpu.sync_copy(acc_sh.at[pl.ds(nbase + 3072, 128)],
                    zb.at[pl.ds(0, 128)])
    pltpu.sync_copy(zb.at[pl.ds(0, 128)],
                    agg.at[cid, pl.ds(nbase + 3072, 128)])


def _agg_stage(src2, dst2, norm2, ha, hb):
    f = pl.kernel(
        _k4_body,
        out_type=jax.ShapeDtypeStruct((NC, NPAD, 32), jnp.float32),
        mesh=_mesh,
        scratch_types=[
            pltpu.VMEM((8, 128), jnp.int32),
            pltpu.VMEM((8, 128), jnp.int32),
            pltpu.VMEM((8, 128), jnp.float32),
            pltpu.VMEM((8, 128, 32), jnp.float32),
            pltpu.VMEM((1024, 32), jnp.float32),
            pltpu.VMEM_SHARED((NPAD, 32), jnp.float32),
            pltpu.SemaphoreType.DMA,
        ],
    )
    return f(src2, dst2, norm2, ha, hb)


def _k5_body(agg_ref, ha_ref, hb_ref, dis_ref, cp_ref, b1_ref, w2t_ref,
             b2_ref, out_ref, acc_ref):
    i = pl.program_id(0)
    d = dis_ref[...]
    inv = d * d
    cvec = cp_ref[0] + cp_ref[1] + inv
    b1 = b1_ref[...]
    a = agg_ref[...]
    ga = jnp.maximum(a[0] + ha_ref[...] * inv[:, None] + b1[0, :32], 0.0)
    gb = jnp.maximum(a[1] + hb_ref[...] * inv[:, None] + b1[0, 32:], 0.0)
    pa = jnp.dot(cvec[None, :], ga, preferred_element_type=jnp.float32)
    pb = jnp.dot(cvec[None, :], gb, preferred_element_type=jnp.float32)
    part = jnp.concatenate([pa, pb], axis=1)

    @pl.when(i == 0)
    def _():
        acc_ref[...] = jnp.zeros_like(acc_ref)

    acc_ref[...] += part

    @pl.when(i == pl.num_programs(0) - 1)
    def _():
        out_ref[...] = (
            jnp.dot(acc_ref[...] * (1.0 / NN), w2t_ref[...],
                    preferred_element_type=jnp.float32)
            + b2_ref[...])


def _finish_stage(agg, ha, hb, dis, cp, b1, w2t, b2):
    return pl.pallas_call(
        _k5_body,
        grid=(GRID_N,),
        in_specs=[
            pl.BlockSpec((2, BN, 32), lambda i: (0, i, 0)),
            pl.BlockSpec((BN, 32), lambda i: (i, 0)),
            pl.BlockSpec((BN, 32), lambda i: (i, 0)),
            pl.BlockSpec((BN,), lambda i: (i,)),
            pl.BlockSpec((2, BN), lambda i: (0, i)),
            pl.BlockSpec((1, 64), lambda i: (0, 0)),
            pl.BlockSpec((64, 64), lambda i: (0, 0)),
            pl.BlockSpec((1, 64), lambda i: (0, 0)),
        ],
        out_specs=pl.BlockSpec((1, 64), lambda i: (0, 0)),
        out_shape=jax.ShapeDtypeStruct((1, 64), jnp.float32),
        scratch_shapes=[pltpu.VMEM((1, 64), jnp.float32)],
    )(agg, ha, hb, dis, cp, b1, w2t, b2)


def _band_matrices(conv1_w, conv2_w):
    w1 = conv1_w[0, 0]
    w2 = conv2_w[0, 0]
    k = jnp.arange(KW)
    s = jnp.arange(S1)
    t = jnp.arange(S2)
    a1f = jnp.zeros((TT, S1), jnp.float32).at[
        2 * (s[None, :] + k[:, None]), jnp.broadcast_to(s[None, :], (KW, S1))
    ].add(jnp.broadcast_to(w1[:, None], (KW, S1)))
    a2 = jnp.zeros((S1, S2), jnp.float32).at[
        t[None, :] + k[:, None], jnp.broadcast_to(t[None, :], (KW, S2))
    ].add(jnp.broadcast_to(w2[:, None], (KW, S2)))
    return a1f, a2


def kernel(node_features, edge_index, edge_attributes, conv1_w, conv1_b,
           conv2_w, conv2_b, W1, b1, W2, b2):
    src = edge_index[0].astype(jnp.int32)
    dst = edge_index[1].astype(jnp.int32)
    ew = edge_attributes.astype(jnp.float32)
    npadlen = EPAD - E0
    pad_idx = jnp.full((npadlen,), NN - 1, jnp.int32)
    src2 = jnp.concatenate([src, pad_idx]).reshape(EROWS, 128)
    dst2 = jnp.concatenate([dst, pad_idx]).reshape(EROWS, 128)
    ew2 = jnp.concatenate(
        [ew, jnp.zeros((npadlen,), jnp.float32)]).reshape(EROWS, 128)

    a1f, a2 = _band_matrices(conv1_w, conv2_w)
    sb = jnp.stack([conv1_b[0], conv2_b[0]]).reshape(1, 2)

    ha, hb = _dense_stage(node_features, a1f, a2, W1.T, sb)
    degp = _deg_stage(dst2, ew2)
    dis = _dis_stage(degp)
    norm2, cp = _norm_stage(src2, dst2, ew2, dis)
    agg = _agg_stage(src2, dst2, norm2, ha, hb)
    out = _finish_stage(agg, ha, hb, dis, cp, b1.reshape(1, 64), W2.T,
                        b2.reshape(1, 64))
    return out


# speedup vs baseline: 12.9508x; 12.9508x over previous
import functools
import jax
import jax.numpy as jnp
import numpy as np
from jax import lax
from jax.experimental import pallas as pl
from jax.experimental.pallas import tpu as pltpu
from jax.experimental.pallas import tpu_sc as plsc

NN = 50000
TT = 518
S1 = 114
S2 = 83
KW = 32
E0 = 800000
EPAD = 819200
EROWS = EPAD // 128
NPAD = 51200
NC, NS = 2, 16
BN = 2000
GRID_N = NN // BN
BN5 = 2048

@functools.lru_cache(maxsize=1)
def _mesh():
    return plsc.VectorSubcoreMesh(
        core_axis_name="c", subcore_axis_name="s",
        num_cores=NC, num_subcores=NS)


def _k1_body(x_ref, a1_ref, a2_ref, w1t_ref, sb_ref, h0_ref, h1_ref,
             h2_ref, h3_ref):
    x = x_ref[...]
    t1 = jnp.maximum(
        jnp.dot(x, a1_ref[...], preferred_element_type=jnp.float32)
        + sb_ref[0, 0], 0.0)
    t2 = jnp.maximum(
        jnp.dot(t1, a2_ref[...], preferred_element_type=jnp.float32)
        + sb_ref[0, 1], 0.0)
    h = jnp.dot(t2, w1t_ref[...], preferred_element_type=jnp.float32)
    h0_ref[...] = h[:, 0:16]
    h1_ref[...] = h[:, 16:32]
    h2_ref[...] = h[:, 32:48]
    h3_ref[...] = h[:, 48:64]


def _dense_stage(nf, a1f, a2, w1t, sb):
    return pl.pallas_call(
        _k1_body,
        grid=(GRID_N,),
        in_specs=[
            pl.BlockSpec((BN, TT), lambda i: (i, 0)),
            pl.BlockSpec((TT, S1), lambda i: (0, 0)),
            pl.BlockSpec((S1, S2), lambda i: (0, 0)),
            pl.BlockSpec((S2, 64), lambda i: (0, 0)),
            pl.BlockSpec((1, 2), lambda i: (0, 0)),
        ],
        out_specs=[
            pl.BlockSpec((BN, 16), lambda i: (i, 0)) for _ in range(4)
        ],
        out_shape=[
            jax.ShapeDtypeStruct((NPAD, 16), jnp.float32) for _ in range(4)
        ],
    )(nf, a1f, a2, w1t, sb)


def _dis_body(dp_ref, dis_ref):
    p = dp_ref[...]
    dis_ref[...] = lax.rsqrt(1.0 + p[0] + p[1])


def _dis_stage(degp):
    return pl.pallas_call(
        _dis_body,
        grid=(1,),
        in_specs=[pl.BlockSpec((2, NPAD), lambda i: (0, 0))],
        out_specs=pl.BlockSpec((NPAD,), lambda i: (0,)),
        out_shape=jax.ShapeDtypeStruct((NPAD,), jnp.float32),
    )(degp)


def _zero_1d(ref, n):
    def body(i, _):
        ref[pl.ds(i * 16, 16)] = jnp.zeros((16,), jnp.float32)
        return 0
    lax.fori_loop(0, n // 16, body, 0)


def _k2_body(dst2, ew2, degp, idx_v, val_v, stripe_v, acc_sh, sem):
    cid = lax.axis_index("c")
    sid = lax.axis_index("s")
    wid = sid * NC + cid
    nbase = pl.multiple_of(sid * (NPAD // NS), 8)

    _zero_1d(stripe_v, NPAD // NS)
    pltpu.sync_copy(stripe_v, acc_sh.at[pl.ds(nbase, NPAD // NS)])
    plsc.subcore_barrier()

    rows_per_tile = EROWS // (NC * NS)
    rowbase = wid * rows_per_tile

    def chunk(ch, _):
        r0 = rowbase + ch * 8
        pltpu.sync_copy(dst2.at[pl.ds(r0, 8)], idx_v)
        pltpu.sync_copy(ew2.at[pl.ds(r0, 8)], val_v)
        for j in range(8):
            pltpu.sync_copy(val_v.at[j], acc_sh.at[idx_v.at[j]], add=True)
        return 0
    lax.fori_loop(0, rows_per_tile // 8, chunk, 0)

    plsc.subcore_barrier()
    pltpu.sync_copy(acc_sh.at[pl.ds(nbase, NPAD // NS)], stripe_v)
    pltpu.sync_copy(stripe_v, degp.at[cid, pl.ds(nbase, NPAD // NS)])


def _deg_stage(dst2, ew2):
    f = pl.kernel(
        _k2_body,
        out_type=jax.ShapeDtypeStruct((NC, NPAD), jnp.float32),
        mesh=_mesh(),
        compiler_params=pltpu.CompilerParams(use_tc_tiling_on_sc=False),
        scratch_types=[
            pltpu.VMEM((8, 128), jnp.int32),
            pltpu.VMEM((8, 128), jnp.float32),
            pltpu.VMEM((NPAD // NS,), jnp.float32),
            pltpu.VMEM_SHARED((NPAD,), jnp.float32),
            pltpu.SemaphoreType.DMA,
        ],
    )
    return f(dst2, ew2)


def _k3_body(src2, dst2, ew2, dis, norm2, cp, sidx, didx, ewv, dsv, ddv, nv,
             stripe_v, acc_sh, sem):
    cid = lax.axis_index("c")
    sid = lax.axis_index("s")
    wid = sid * NC + cid
    nbase = pl.multiple_of(sid * (NPAD // NS), 8)

    _zero_1d(stripe_v, NPAD // NS)
    pltpu.sync_copy(stripe_v, acc_sh.at[pl.ds(nbase, NPAD // NS)])
    plsc.subcore_barrier()

    rows_per_tile = EROWS // (NC * NS)
    rowbase = wid * rows_per_tile

    def chunk(ch, _):
        r0 = rowbase + ch * 8
        pltpu.sync_copy(src2.at[pl.ds(r0, 8)], sidx)
        pltpu.sync_copy(dst2.at[pl.ds(r0, 8)], didx)
        pltpu.sync_copy(ew2.at[pl.ds(r0, 8)], ewv)
        for j in range(8):
            pltpu.sync_copy(dis.at[sidx.at[j]], dsv.at[j])
            pltpu.sync_copy(dis.at[didx.at[j]], ddv.at[j])
        for j in range(8):
            for i in range(8):
                sl = pl.ds(i * 16, 16)
                nv[j, sl] = dsv[j, sl] * ewv[j, sl] * ddv[j, sl]
        pltpu.sync_copy(nv, norm2.at[pl.ds(r0, 8)])
        for j in range(8):
            pltpu.sync_copy(nv.at[j], acc_sh.at[sidx.at[j]], add=True)
        return 0
    lax.fori_loop(0, rows_per_tile // 8, chunk, 0)

    plsc.subcore_barrier()
    pltpu.sync_copy(acc_sh.at[pl.ds(nbase, NPAD // NS)], stripe_v)
    pltpu.sync_copy(stripe_v, cp.at[cid, pl.ds(nbase, NPAD // NS)])


def _norm_stage(src2, dst2, ew2, dis):
    f = pl.kernel(
        _k3_body,
        out_type=(
            jax.ShapeDtypeStruct((EROWS, 128), jnp.float32),
            jax.ShapeDtypeStruct((NC, NPAD), jnp.float32),
        ),
        mesh=_mesh(),
        compiler_params=pltpu.CompilerParams(use_tc_tiling_on_sc=False),
        scratch_types=[
            pltpu.VMEM((8, 128), jnp.int32),
            pltpu.VMEM((8, 128), jnp.int32),
            pltpu.VMEM((8, 128), jnp.float32),
            pltpu.VMEM((8, 128), jnp.float32),
            pltpu.VMEM((8, 128), jnp.float32),
            pltpu.VMEM((8, 128), jnp.float32),
            pltpu.VMEM((NPAD // NS,), jnp.float32),
            pltpu.VMEM_SHARED((NPAD,), jnp.float32),
            pltpu.SemaphoreType.DMA,
        ],
    )
    return f(src2, dst2, ew2, dis)


def _k4_body(src2, dst2, norm2, h0, h1, h2, h3, agg, sidx, didx, nv, rows_v,
             zb, acc_sh, sem):
    cid = lax.axis_index("c")
    sid = lax.axis_index("s")
    nbase = pl.multiple_of(sid * (NPAD // NS), 8)

    def zb_zero(i, _):
        zb[i, pl.ds(0, 16)] = jnp.zeros((16,), jnp.float32)
        return 0

    def zero_acc():
        lax.fori_loop(0, 1024, zb_zero, 0)
        for t in range(3):
            pltpu.sync_copy(zb, acc_sh.at[pl.ds(nbase + t * 1024, 1024)])
        pltpu.sync_copy(zb.at[pl.ds(0, 128)],
                        acc_sh.at[pl.ds(nbase + 3072, 128)])

    def dump_acc(q):
        for t in range(3):
            pltpu.sync_copy(acc_sh.at[pl.ds(nbase + t * 1024, 1024)], zb)
            pltpu.sync_copy(zb, agg.at[q, pl.ds(nbase + t * 1024, 1024)])
        pltpu.sync_copy(acc_sh.at[pl.ds(nbase + 3072, 128)],
                        zb.at[pl.ds(0, 128)])
        pltpu.sync_copy(zb.at[pl.ds(0, 128)],
                        agg.at[q, pl.ds(nbase + 3072, 128)])

    rows_per_tile = EROWS // NS
    rowbase = sid * rows_per_tile

    def sweep(table):
        def chunk(ch, _):
            r0 = rowbase + ch * 8
            pltpu.sync_copy(src2.at[pl.ds(r0, 8)], sidx)
            pltpu.sync_copy(dst2.at[pl.ds(r0, 8)], didx)
            pltpu.sync_copy(norm2.at[pl.ds(r0, 8)], nv)
            for j in range(8):
                pltpu.sync_copy(table.at[sidx.at[j]], rows_v.at[j])
            for j in range(8):
                def scale16(i, _, j=j):
                    nvec = nv[j, pl.ds(i * 16, 16)]
                    base = i * 16
                    for l in range(16):
                        n = nvec[l]
                        e = base + l
                        rows_v[j, e, pl.ds(0, 16)] = (
                            rows_v[j, e, pl.ds(0, 16)] * n)
                    return 0
                lax.fori_loop(0, 8, scale16, 0)
            for j in range(8):
                pltpu.sync_copy(rows_v.at[j], acc_sh.at[didx.at[j]],
                                add=True)
            return 0
        lax.fori_loop(0, rows_per_tile // 8, chunk, 0)

    def run_quarters(ta, tb, qa, qb):
        zero_acc()
        plsc.subcore_barrier()
        sweep(ta)
        plsc.subcore_barrier()
        dump_acc(qa)
        zero_acc()
        plsc.subcore_barrier()
        sweep(tb)
        plsc.subcore_barrier()
        dump_acc(qb)

    @pl.when(cid == 0)
    def _():
        run_quarters(h0, h1, 0, 1)

    @pl.when(cid == 1)
    def _():
        run_quarters(h2, h3, 2, 3)


def _agg_stage(src2, dst2, norm2, h0, h1, h2, h3):
    f = pl.kernel(
        _k4_body,
        out_type=jax.ShapeDtypeStruct((4, NPAD, 16), jnp.float32),
        mesh=_mesh(),
        compiler_params=pltpu.CompilerParams(use_tc_tiling_on_sc=False),
        scratch_types=[
            pltpu.VMEM((8, 128), jnp.int32),
            pltpu.VMEM((8, 128), jnp.int32),
            pltpu.VMEM((8, 128), jnp.float32),
            pltpu.VMEM((8, 128, 16), jnp.float32),
            pltpu.VMEM((1024, 16), jnp.float32),
            pltpu.VMEM_SHARED((NPAD, 16), jnp.float32),
            pltpu.SemaphoreType.DMA,
        ],
    )
    return f(src2, dst2, norm2, h0, h1, h2, h3)


def _k5_body(agg_ref, h0_ref, h1_ref, h2_ref, h3_ref, dis_ref, cp_ref,
             b1_ref, w2t_ref, b2_ref, out_ref, acc_ref):
    i = pl.program_id(0)
    d = dis_ref[...]
    inv = d * d
    cvec = cp_ref[0] + cp_ref[1] + inv
    b1 = b1_ref[...]
    a = agg_ref[...]
    rowid = jax.lax.broadcasted_iota(jnp.int32, (BN5, 1), 0) + i * BN5
    valid = rowid < NN
    hq = [h0_ref[...], h1_ref[...], h2_ref[...], h3_ref[...]]
    parts = []
    for q in range(4):
        gq = jnp.maximum(
            a[q] + hq[q] * inv[:, None] + b1[0, 16 * q:16 * q + 16], 0.0)
        gq = jnp.where(valid, gq, 0.0)
        parts.append(
            jnp.dot(cvec[None, :], gq, preferred_element_type=jnp.float32))
    part = jnp.concatenate(parts, axis=1)

    @pl.when(i == 0)
    def _():
        acc_ref[...] = jnp.zeros_like(acc_ref)

    acc_ref[...] += part

    @pl.when(i == pl.num_programs(0) - 1)
    def _():
        out_ref[...] = (
            jnp.dot(acc_ref[...] * (1.0 / NN), w2t_ref[...],
                    preferred_element_type=jnp.float32)
            + b2_ref[...])


def _finish_stage(agg, h0, h1, h2, h3, dis, cp, b1, w2t, b2):
    return pl.pallas_call(
        _k5_body,
        grid=(NPAD // BN5,),
        in_specs=[
            pl.BlockSpec((4, BN5, 16), lambda i: (0, i, 0)),
            pl.BlockSpec((BN5, 16), lambda i: (i, 0)),
            pl.BlockSpec((BN5, 16), lambda i: (i, 0)),
            pl.BlockSpec((BN5, 16), lambda i: (i, 0)),
            pl.BlockSpec((BN5, 16), lambda i: (i, 0)),
            pl.BlockSpec((BN5,), lambda i: (i,)),
            pl.BlockSpec((2, BN5), lambda i: (0, i)),
            pl.BlockSpec((1, 64), lambda i: (0, 0)),
            pl.BlockSpec((64, 64), lambda i: (0, 0)),
            pl.BlockSpec((1, 64), lambda i: (0, 0)),
        ],
        out_specs=pl.BlockSpec((1, 64), lambda i: (0, 0)),
        out_shape=jax.ShapeDtypeStruct((1, 64), jnp.float32),
        scratch_shapes=[pltpu.VMEM((1, 64), jnp.float32)],
    )(agg, h0, h1, h2, h3, dis, cp, b1, w2t, b2)


def _band_matrices(conv1_w, conv2_w):
    w1 = conv1_w[0, 0]
    w2 = conv2_w[0, 0]
    k = jnp.arange(KW)
    s = jnp.arange(S1)
    t = jnp.arange(S2)
    a1f = jnp.zeros((TT, S1), jnp.float32).at[
        4 * s[None, :] + 2 * k[:, None], jnp.broadcast_to(s[None, :], (KW, S1))
    ].add(jnp.broadcast_to(w1[:, None], (KW, S1)))
    a2 = jnp.zeros((S1, S2), jnp.float32).at[
        t[None, :] + k[:, None], jnp.broadcast_to(t[None, :], (KW, S2))
    ].add(jnp.broadcast_to(w2[:, None], (KW, S2)))
    return a1f, a2


def kernel(node_features, edge_index, edge_attributes, conv1_w, conv1_b,
           conv2_w, conv2_b, W1, b1, W2, b2):
    src = edge_index[0].astype(jnp.int32)
    dst = edge_index[1].astype(jnp.int32)
    ew = edge_attributes.astype(jnp.float32)
    npadlen = EPAD - E0
    pad_idx = jnp.full((npadlen,), NN - 1, jnp.int32)
    src2 = jnp.concatenate([src, pad_idx]).reshape(EROWS, 128)
    dst2 = jnp.concatenate([dst, pad_idx]).reshape(EROWS, 128)
    ew2 = jnp.concatenate(
        [ew, jnp.zeros((npadlen,), jnp.float32)]).reshape(EROWS, 128)

    a1f, a2 = _band_matrices(conv1_w, conv2_w)
    sb = jnp.stack([conv1_b[0], conv2_b[0]]).reshape(1, 2)

    h0, h1, h2, h3 = _dense_stage(node_features, a1f, a2, W1.T, sb)
    degp = _deg_stage(dst2, ew2)
    dis = _dis_stage(degp)
    norm2, cp = _norm_stage(src2, dst2, ew2, dis)
    agg = _agg_stage(src2, dst2, norm2, h0, h1, h2, h3)
    out = _finish_stage(agg, h0, h1, h2, h3, dis, cp, b1.reshape(1, 64),
                        W2.T, b2.reshape(1, 64))
    return out


# speedup vs baseline: 14.7942x; 1.1423x over previous
import functools
import jax
import jax.numpy as jnp
import numpy as np
from jax import lax
from jax.experimental import pallas as pl
from jax.experimental.pallas import tpu as pltpu
from jax.experimental.pallas import tpu_sc as plsc

NN = 50000
TT = 518
S1 = 114
S2 = 83
KW = 32
E0 = 800000
EPAD = 819200
EROWS = EPAD // 128
NPAD = 51200
NC, NS = 2, 16
BN = 2000
GRID_N = NN // BN
BN5 = 2048

@functools.lru_cache(maxsize=1)
def _mesh():
    return plsc.VectorSubcoreMesh(
        core_axis_name="c", subcore_axis_name="s",
        num_cores=NC, num_subcores=NS)


def _k1_body(x_ref, a1_ref, a2_ref, w1t_ref, sb_ref, h0_ref, h1_ref,
             h2_ref, h3_ref):
    x = x_ref[...]
    t1 = jnp.maximum(
        jnp.dot(x, a1_ref[...], preferred_element_type=jnp.float32)
        + sb_ref[0, 0], 0.0)
    t2 = jnp.maximum(
        jnp.dot(t1, a2_ref[...], preferred_element_type=jnp.float32)
        + sb_ref[0, 1], 0.0)
    h = jnp.dot(t2, w1t_ref[...], preferred_element_type=jnp.float32)
    h0_ref[...] = h[:, 0:16]
    h1_ref[...] = h[:, 16:32]
    h2_ref[...] = h[:, 32:48]
    h3_ref[...] = h[:, 48:64]


def _dense_stage(nf, a1f, a2, w1t, sb):
    return pl.pallas_call(
        _k1_body,
        grid=(GRID_N,),
        in_specs=[
            pl.BlockSpec((BN, TT), lambda i: (i, 0)),
            pl.BlockSpec((TT, S1), lambda i: (0, 0)),
            pl.BlockSpec((S1, S2), lambda i: (0, 0)),
            pl.BlockSpec((S2, 64), lambda i: (0, 0)),
            pl.BlockSpec((1, 2), lambda i: (0, 0)),
        ],
        out_specs=[
            pl.BlockSpec((BN, 16), lambda i: (i, 0)) for _ in range(4)
        ],
        out_shape=[
            jax.ShapeDtypeStruct((NPAD, 16), jnp.float32) for _ in range(4)
        ],
    )(nf, a1f, a2, w1t, sb)


def _dis_body(dp_ref, dis_ref):
    p = dp_ref[...]
    dis_ref[...] = lax.rsqrt(1.0 + p[0] + p[1])


def _dis_stage(degp):
    return pl.pallas_call(
        _dis_body,
        grid=(1,),
        in_specs=[pl.BlockSpec((2, NPAD), lambda i: (0, 0))],
        out_specs=pl.BlockSpec((NPAD,), lambda i: (0,)),
        out_shape=jax.ShapeDtypeStruct((NPAD,), jnp.float32),
    )(degp)


def _zero_1d(ref, n):
    def body(i, _):
        ref[pl.ds(i * 16, 16)] = jnp.zeros((16,), jnp.float32)
        return 0
    lax.fori_loop(0, n // 16, body, 0)


def _k2_body(dst2, ew2, degp, idx_v, val_v, stripe_v, acc_sh, sem):
    cid = lax.axis_index("c")
    sid = lax.axis_index("s")
    wid = sid * NC + cid
    nbase = pl.multiple_of(sid * (NPAD // NS), 8)

    _zero_1d(stripe_v, NPAD // NS)
    pltpu.sync_copy(stripe_v, acc_sh.at[pl.ds(nbase, NPAD // NS)])
    plsc.subcore_barrier()

    rows_per_tile = EROWS // (NC * NS)
    rowbase = wid * rows_per_tile

    def chunk(ch, _):
        r0 = rowbase + ch * 8
        pltpu.sync_copy(dst2.at[pl.ds(r0, 8)], idx_v)
        pltpu.sync_copy(ew2.at[pl.ds(r0, 8)], val_v)
        for j in range(8):
            pltpu.sync_copy(val_v.at[j], acc_sh.at[idx_v.at[j]], add=True)
        return 0
    lax.fori_loop(0, rows_per_tile // 8, chunk, 0)

    plsc.subcore_barrier()
    pltpu.sync_copy(acc_sh.at[pl.ds(nbase, NPAD // NS)], stripe_v)
    pltpu.sync_copy(stripe_v, degp.at[cid, pl.ds(nbase, NPAD // NS)])


def _deg_stage(dst2, ew2):
    f = pl.kernel(
        _k2_body,
        out_type=jax.ShapeDtypeStruct((NC, NPAD), jnp.float32),
        mesh=_mesh(),
        compiler_params=pltpu.CompilerParams(use_tc_tiling_on_sc=False),
        scratch_types=[
            pltpu.VMEM((8, 128), jnp.int32),
            pltpu.VMEM((8, 128), jnp.float32),
            pltpu.VMEM((NPAD // NS,), jnp.float32),
            pltpu.VMEM_SHARED((NPAD,), jnp.float32),
            pltpu.SemaphoreType.DMA,
        ],
    )
    return f(dst2, ew2)


def _k3_body(src2, dst2, ew2, dis, norm2, cp, sidx, didx, ewv, dsv, ddv, nv,
             stripe_v, acc_sh, sem):
    cid = lax.axis_index("c")
    sid = lax.axis_index("s")
    wid = sid * NC + cid
    nbase = pl.multiple_of(sid * (NPAD // NS), 8)

    _zero_1d(stripe_v, NPAD // NS)
    pltpu.sync_copy(stripe_v, acc_sh.at[pl.ds(nbase, NPAD // NS)])
    plsc.subcore_barrier()

    rows_per_tile = EROWS // (NC * NS)
    rowbase = wid * rows_per_tile

    def chunk(ch, _):
        r0 = rowbase + ch * 8
        pltpu.sync_copy(src2.at[pl.ds(r0, 8)], sidx)
        pltpu.sync_copy(dst2.at[pl.ds(r0, 8)], didx)
        pltpu.sync_copy(ew2.at[pl.ds(r0, 8)], ewv)
        for j in range(8):
            pltpu.sync_copy(dis.at[sidx.at[j]], dsv.at[j])
            pltpu.sync_copy(dis.at[didx.at[j]], ddv.at[j])
        for j in range(8):
            for i in range(8):
                sl = pl.ds(i * 16, 16)
                nv[j, sl] = dsv[j, sl] * ewv[j, sl] * ddv[j, sl]
        pltpu.sync_copy(nv, norm2.at[pl.ds(r0, 8)])
        for j in range(8):
            pltpu.sync_copy(nv.at[j], acc_sh.at[sidx.at[j]], add=True)
        return 0
    lax.fori_loop(0, rows_per_tile // 8, chunk, 0)

    plsc.subcore_barrier()
    pltpu.sync_copy(acc_sh.at[pl.ds(nbase, NPAD // NS)], stripe_v)
    pltpu.sync_copy(stripe_v, cp.at[cid, pl.ds(nbase, NPAD // NS)])


def _norm_stage(src2, dst2, ew2, dis):
    f = pl.kernel(
        _k3_body,
        out_type=(
            jax.ShapeDtypeStruct((EROWS, 128), jnp.float32),
            jax.ShapeDtypeStruct((NC, NPAD), jnp.float32),
        ),
        mesh=_mesh(),
        compiler_params=pltpu.CompilerParams(use_tc_tiling_on_sc=False),
        scratch_types=[
            pltpu.VMEM((8, 128), jnp.int32),
            pltpu.VMEM((8, 128), jnp.int32),
            pltpu.VMEM((8, 128), jnp.float32),
            pltpu.VMEM((8, 128), jnp.float32),
            pltpu.VMEM((8, 128), jnp.float32),
            pltpu.VMEM((8, 128), jnp.float32),
            pltpu.VMEM((NPAD // NS,), jnp.float32),
            pltpu.VMEM_SHARED((NPAD,), jnp.float32),
            pltpu.SemaphoreType.DMA,
        ],
    )
    return f(src2, dst2, ew2, dis)


def _k4_body(srcl, dstl, norml, hflat, agg, sidx, didx, nv, rows_v, zb,
             acc_sh, sem):
    cid = lax.axis_index("c")
    sid = lax.axis_index("s")
    nbase = pl.multiple_of(sid * (NPAD // NS), 8)

    def zb_zero(i, _):
        zb[i, pl.ds(0, 16)] = jnp.zeros((16,), jnp.float32)
        return 0

    def zero_acc():
        lax.fori_loop(0, 1024, zb_zero, 0)
        for t in range(3):
            pltpu.sync_copy(zb, acc_sh.at[pl.ds(nbase + t * 1024, 1024)])
        pltpu.sync_copy(zb.at[pl.ds(0, 128)],
                        acc_sh.at[pl.ds(nbase + 3072, 128)])

    def dump_acc(q):
        for t in range(3):
            pltpu.sync_copy(acc_sh.at[pl.ds(nbase + t * 1024, 1024)], zb)
            pltpu.sync_copy(zb, agg.at[q, pl.ds(nbase + t * 1024, 1024)])
        pltpu.sync_copy(acc_sh.at[pl.ds(nbase + 3072, 128)],
                        zb.at[pl.ds(0, 128)])
        pltpu.sync_copy(zb.at[pl.ds(0, 128)],
                        agg.at[q, pl.ds(nbase + 3072, 128)])

    edges_per_tile = EPAD // NS
    ebase = sid * edges_per_tile
    nch = edges_per_tile // 1024

    def do_pass(p, _):
        q = cid * 2 + p
        qoff = q * NPAD
        zero_acc()
        plsc.subcore_barrier()

        def chunk(ch, _):
            e0 = ebase + ch * 1024
            pltpu.sync_copy(srcl.at[pl.ds(e0, 1024)], sidx)
            pltpu.sync_copy(dstl.at[pl.ds(e0, 1024)], didx)
            pltpu.sync_copy(norml.at[pl.ds(e0, 1024)], nv)

            def addq(i, _):
                sl = pl.ds(i * 16, 16)
                sidx[sl] = sidx[sl] + qoff
                return 0
            lax.fori_loop(0, 64, addq, 0)
            pltpu.sync_copy(hflat.at[sidx], rows_v)

            def scale16(i, _):
                nvec = nv[pl.ds(i * 16, 16)]
                base = i * 16
                for l in range(16):
                    rows_v[base + l, pl.ds(0, 16)] = (
                        rows_v[base + l, pl.ds(0, 16)] * nvec[l])
                return 0
            lax.fori_loop(0, 64, scale16, 0)
            pltpu.sync_copy(rows_v, acc_sh.at[didx], add=True)
            return 0
        lax.fori_loop(0, nch, chunk, 0)

        plsc.subcore_barrier()
        dump_acc(q)
        return 0
    lax.fori_loop(0, 2, do_pass, 0)


def _agg_stage(srcl, dstl, norml, hflat):
    f = pl.kernel(
        _k4_body,
        out_type=jax.ShapeDtypeStruct((4, NPAD, 16), jnp.float32),
        mesh=_mesh(),
        compiler_params=pltpu.CompilerParams(use_tc_tiling_on_sc=False),
        scratch_types=[
            pltpu.VMEM((1024,), jnp.int32),
            pltpu.VMEM((1024,), jnp.int32),
            pltpu.VMEM((1024,), jnp.float32),
            pltpu.VMEM((1024, 16), jnp.float32),
            pltpu.VMEM((1024, 16), jnp.float32),
            pltpu.VMEM_SHARED((NPAD, 16), jnp.float32),
            pltpu.SemaphoreType.DMA,
        ],
    )
    return f(srcl, dstl, norml, hflat)


def _k5_body(agg_ref, h0_ref, h1_ref, h2_ref, h3_ref, dis_ref, cp_ref,
             b1_ref, w2t_ref, b2_ref, out_ref, acc_ref):
    i = pl.program_id(0)
    d = dis_ref[...]
    inv = d * d
    cvec = cp_ref[0] + cp_ref[1] + inv
    b1 = b1_ref[...]
    a = agg_ref[...]
    rowid = jax.lax.broadcasted_iota(jnp.int32, (BN5, 1), 0) + i * BN5
    valid = rowid < NN
    hq = [h0_ref[...], h1_ref[...], h2_ref[...], h3_ref[...]]
    parts = []
    for q in range(4):
        gq = jnp.maximum(
            a[q] + hq[q] * inv[:, None] + b1[0, 16 * q:16 * q + 16], 0.0)
        gq = jnp.where(valid, gq, 0.0)
        parts.append(
            jnp.dot(cvec[None, :], gq, preferred_element_type=jnp.float32))
    part = jnp.concatenate(parts, axis=1)

    @pl.when(i == 0)
    def _():
        acc_ref[...] = jnp.zeros_like(acc_ref)

    acc_ref[...] += part

    @pl.when(i == pl.num_programs(0) - 1)
    def _():
        out_ref[...] = (
            jnp.dot(acc_ref[...] * (1.0 / NN), w2t_ref[...],
                    preferred_element_type=jnp.float32)
            + b2_ref[...])


def _finish_stage(agg, h0, h1, h2, h3, dis, cp, b1, w2t, b2):
    return pl.pallas_call(
        _k5_body,
        grid=(NPAD // BN5,),
        in_specs=[
            pl.BlockSpec((4, BN5, 16), lambda i: (0, i, 0)),
            pl.BlockSpec((BN5, 16), lambda i: (i, 0)),
            pl.BlockSpec((BN5, 16), lambda i: (i, 0)),
            pl.BlockSpec((BN5, 16), lambda i: (i, 0)),
            pl.BlockSpec((BN5, 16), lambda i: (i, 0)),
            pl.BlockSpec((BN5,), lambda i: (i,)),
            pl.BlockSpec((2, BN5), lambda i: (0, i)),
            pl.BlockSpec((1, 64), lambda i: (0, 0)),
            pl.BlockSpec((64, 64), lambda i: (0, 0)),
            pl.BlockSpec((1, 64), lambda i: (0, 0)),
        ],
        out_specs=pl.BlockSpec((1, 64), lambda i: (0, 0)),
        out_shape=jax.ShapeDtypeStruct((1, 64), jnp.float32),
        scratch_shapes=[pltpu.VMEM((1, 64), jnp.float32)],
    )(agg, h0, h1, h2, h3, dis, cp, b1, w2t, b2)


def _band_matrices(conv1_w, conv2_w):
    w1 = conv1_w[0, 0]
    w2 = conv2_w[0, 0]
    i1 = jnp.arange(TT)[:, None]
    s1 = jnp.arange(S1)[None, :]
    d1 = i1 - 4 * s1
    a1f = jnp.zeros((TT, S1), jnp.float32)
    for k in range(KW):
        a1f = a1f + jnp.where(d1 == 2 * k, w1[k], 0.0)
    i2 = jnp.arange(S1)[:, None]
    t2 = jnp.arange(S2)[None, :]
    d2 = i2 - t2
    a2 = jnp.zeros((S1, S2), jnp.float32)
    for k in range(KW):
        a2 = a2 + jnp.where(d2 == k, w2[k], 0.0)
    return a1f, a2


def kernel(node_features, edge_index, edge_attributes, conv1_w, conv1_b,
           conv2_w, conv2_b, W1, b1, W2, b2):
    src = edge_index[0].astype(jnp.int32)
    dst = edge_index[1].astype(jnp.int32)
    ew = edge_attributes.astype(jnp.float32)
    npadlen = EPAD - E0
    pad_idx = jnp.full((npadlen,), NN - 1, jnp.int32)
    src2 = jnp.concatenate([src, pad_idx]).reshape(EROWS, 128)
    dst2 = jnp.concatenate([dst, pad_idx]).reshape(EROWS, 128)
    ew2 = jnp.concatenate(
        [ew, jnp.zeros((npadlen,), jnp.float32)]).reshape(EROWS, 128)

    a1f, a2 = _band_matrices(conv1_w, conv2_w)
    sb = jnp.stack([conv1_b[0], conv2_b[0]]).reshape(1, 2)

    h0, h1, h2, h3 = _dense_stage(node_features, a1f, a2, W1.T, sb)
    degp = _deg_stage(dst2, ew2)
    dis = _dis_stage(degp)
    norm2, cp = _norm_stage(src2, dst2, ew2, dis)
    hflat = jnp.concatenate([h0, h1, h2, h3], axis=0)
    agg = _agg_stage(jnp.concatenate([src, pad_idx]),
                     jnp.concatenate([dst, pad_idx]),
                     norm2.reshape(EPAD), hflat)
    out = _finish_stage(agg, h0, h1, h2, h3, dis, cp, b1.reshape(1, 64),
                        W2.T, b2.reshape(1, 64))
    return out


# speedup vs baseline: 15.9957x; 1.0812x over previous
import functools
import jax
import jax.numpy as jnp
import numpy as np
from jax import lax
from jax.experimental import pallas as pl
from jax.experimental.pallas import tpu as pltpu
from jax.experimental.pallas import tpu_sc as plsc

NN = 50000
TT = 518
S1 = 114
S2 = 83
KW = 32
E0 = 800000
EPAD = 819200
EROWS = EPAD // 128
NPAD = 51200
NC, NS = 2, 16
BN = 2000
GRID_N = NN // BN
BN5 = 2048

@functools.lru_cache(maxsize=1)
def _mesh():
    return plsc.VectorSubcoreMesh(
        core_axis_name="c", subcore_axis_name="s",
        num_cores=NC, num_subcores=NS)


def _k1_body(x_ref, a1_ref, a2_ref, w1t_ref, sb_ref, h0_ref, h1_ref,
             h2_ref, h3_ref):
    x = x_ref[...]
    t1 = jnp.maximum(
        jnp.dot(x, a1_ref[...], preferred_element_type=jnp.float32)
        + sb_ref[0, 0], 0.0)
    t2 = jnp.maximum(
        jnp.dot(t1, a2_ref[...], preferred_element_type=jnp.float32)
        + sb_ref[0, 1], 0.0)
    h = jnp.dot(t2, w1t_ref[...], preferred_element_type=jnp.float32)
    h0_ref[...] = h[:, 0:16]
    h1_ref[...] = h[:, 16:32]
    h2_ref[...] = h[:, 32:48]
    h3_ref[...] = h[:, 48:64]


def _dense_stage(nf, a1f, a2, w1t, sb):
    return pl.pallas_call(
        _k1_body,
        grid=(GRID_N,),
        in_specs=[
            pl.BlockSpec((BN, TT), lambda i: (i, 0)),
            pl.BlockSpec((TT, S1), lambda i: (0, 0)),
            pl.BlockSpec((S1, S2), lambda i: (0, 0)),
            pl.BlockSpec((S2, 64), lambda i: (0, 0)),
            pl.BlockSpec((1, 2), lambda i: (0, 0)),
        ],
        out_specs=[
            pl.BlockSpec((BN, 16), lambda i: (i, 0)) for _ in range(4)
        ],
        out_shape=[
            jax.ShapeDtypeStruct((NPAD, 16), jnp.float32) for _ in range(4)
        ],
    )(nf, a1f, a2, w1t, sb)


def _dis_body(dp_ref, dis_ref):
    p = dp_ref[...]
    dis_ref[...] = lax.rsqrt(1.0 + p[0] + p[1])


def _dis_stage(degp):
    return pl.pallas_call(
        _dis_body,
        grid=(1,),
        in_specs=[pl.BlockSpec((2, NPAD), lambda i: (0, 0))],
        out_specs=pl.BlockSpec((NPAD,), lambda i: (0,)),
        out_shape=jax.ShapeDtypeStruct((NPAD,), jnp.float32),
    )(degp)


def _zero_1d(ref, n):
    def body(i, _):
        ref[pl.ds(i * 16, 16)] = jnp.zeros((16,), jnp.float32)
        return 0
    lax.fori_loop(0, n // 16, body, 0)


def _k2_body(dstl, ewl, degp, idx_v, val_v, stripe_v, acc_sh, sem):
    cid = lax.axis_index("c")
    sid = lax.axis_index("s")
    wid = sid * NC + cid
    nbase = pl.multiple_of(sid * (NPAD // NS), 8)

    _zero_1d(stripe_v, NPAD // NS)
    pltpu.sync_copy(stripe_v, acc_sh.at[pl.ds(nbase, NPAD // NS)])
    plsc.subcore_barrier()

    ept = EPAD // (NC * NS)
    ebase = wid * ept

    def chunk(ch, _):
        e0 = ebase + ch * 1024
        pltpu.sync_copy(dstl.at[pl.ds(e0, 1024)], idx_v)
        pltpu.sync_copy(ewl.at[pl.ds(e0, 1024)], val_v)
        pltpu.sync_copy(val_v, acc_sh.at[idx_v], add=True)
        return 0
    lax.fori_loop(0, ept // 1024, chunk, 0)

    plsc.subcore_barrier()
    pltpu.sync_copy(acc_sh.at[pl.ds(nbase, NPAD // NS)], stripe_v)
    pltpu.sync_copy(stripe_v, degp.at[cid, pl.ds(nbase, NPAD // NS)])


def _deg_stage(dstl, ewl):
    f = pl.kernel(
        _k2_body,
        out_type=jax.ShapeDtypeStruct((NC, NPAD), jnp.float32),
        mesh=_mesh(),
        compiler_params=pltpu.CompilerParams(use_tc_tiling_on_sc=False),
        scratch_types=[
            pltpu.VMEM((1024,), jnp.int32),
            pltpu.VMEM((1024,), jnp.float32),
            pltpu.VMEM((NPAD // NS,), jnp.float32),
            pltpu.VMEM_SHARED((NPAD,), jnp.float32),
            pltpu.SemaphoreType.DMA,
        ],
    )
    return f(dstl, ewl)


def _k3_body(srcl, dstl, ewl, dis, norml, cp, sidx, didx, ewv, dsv, ddv, nv,
             stripe_v, acc_sh, sem):
    cid = lax.axis_index("c")
    sid = lax.axis_index("s")
    wid = sid * NC + cid
    nbase = pl.multiple_of(sid * (NPAD // NS), 8)

    _zero_1d(stripe_v, NPAD // NS)
    pltpu.sync_copy(stripe_v, acc_sh.at[pl.ds(nbase, NPAD // NS)])
    plsc.subcore_barrier()

    ept = EPAD // (NC * NS)
    ebase = wid * ept

    def chunk(ch, _):
        e0 = ebase + ch * 1024
        pltpu.sync_copy(srcl.at[pl.ds(e0, 1024)], sidx)
        pltpu.sync_copy(dstl.at[pl.ds(e0, 1024)], didx)
        pltpu.sync_copy(ewl.at[pl.ds(e0, 1024)], ewv)
        pltpu.sync_copy(dis.at[sidx], dsv)
        pltpu.sync_copy(dis.at[didx], ddv)

        def mul16(i, _):
            sl = pl.ds(i * 16, 16)
            nv[sl] = dsv[sl] * ewv[sl] * ddv[sl]
            return 0
        lax.fori_loop(0, 64, mul16, 0)
        pltpu.sync_copy(nv, norml.at[pl.ds(e0, 1024)])
        pltpu.sync_copy(nv, acc_sh.at[sidx], add=True)
        return 0
    lax.fori_loop(0, ept // 1024, chunk, 0)

    plsc.subcore_barrier()
    pltpu.sync_copy(acc_sh.at[pl.ds(nbase, NPAD // NS)], stripe_v)
    pltpu.sync_copy(stripe_v, cp.at[cid, pl.ds(nbase, NPAD // NS)])


def _norm_stage(srcl, dstl, ewl, dis):
    f = pl.kernel(
        _k3_body,
        out_type=(
            jax.ShapeDtypeStruct((EPAD,), jnp.float32),
            jax.ShapeDtypeStruct((NC, NPAD), jnp.float32),
        ),
        mesh=_mesh(),
        compiler_params=pltpu.CompilerParams(use_tc_tiling_on_sc=False),
        scratch_types=[
            pltpu.VMEM((1024,), jnp.int32),
            pltpu.VMEM((1024,), jnp.int32),
            pltpu.VMEM((1024,), jnp.float32),
            pltpu.VMEM((1024,), jnp.float32),
            pltpu.VMEM((1024,), jnp.float32),
            pltpu.VMEM((1024,), jnp.float32),
            pltpu.VMEM((NPAD // NS,), jnp.float32),
            pltpu.VMEM_SHARED((NPAD,), jnp.float32),
            pltpu.SemaphoreType.DMA,
        ],
    )
    return f(srcl, dstl, ewl, dis)


def _k4_body(srcl, dstl, norml, hflat, agg, sidx, didx, nv, rows_v, zb,
             acc_sh, sem):
    cid = lax.axis_index("c")
    sid = lax.axis_index("s")
    nbase = pl.multiple_of(sid * (NPAD // NS), 8)

    def zb_zero(i, _):
        zb[i, pl.ds(0, 16)] = jnp.zeros((16,), jnp.float32)
        return 0

    def zero_acc():
        lax.fori_loop(0, 1024, zb_zero, 0)
        for t in range(3):
            pltpu.sync_copy(zb, acc_sh.at[pl.ds(nbase + t * 1024, 1024)])
        pltpu.sync_copy(zb.at[pl.ds(0, 128)],
                        acc_sh.at[pl.ds(nbase + 3072, 128)])

    def dump_acc(q):
        for t in range(3):
            pltpu.sync_copy(acc_sh.at[pl.ds(nbase + t * 1024, 1024)], zb)
            pltpu.sync_copy(zb, agg.at[q, pl.ds(nbase + t * 1024, 1024)])
        pltpu.sync_copy(acc_sh.at[pl.ds(nbase + 3072, 128)],
                        zb.at[pl.ds(0, 128)])
        pltpu.sync_copy(zb.at[pl.ds(0, 128)],
                        agg.at[q, pl.ds(nbase + 3072, 128)])

    edges_per_tile = EPAD // NS
    ebase = sid * edges_per_tile
    nch = edges_per_tile // 1024

    def do_pass(p, _):
        q = cid * 2 + p
        qoff = q * NPAD
        zero_acc()
        plsc.subcore_barrier()

        def chunk(ch, _):
            e0 = ebase + ch * 1024
            pltpu.sync_copy(srcl.at[pl.ds(e0, 1024)], sidx)
            pltpu.sync_copy(dstl.at[pl.ds(e0, 1024)], didx)
            pltpu.sync_copy(norml.at[pl.ds(e0, 1024)], nv)

            def addq(i, _):
                sl = pl.ds(i * 16, 16)
                sidx[sl] = sidx[sl] + qoff
                return 0
            lax.fori_loop(0, 64, addq, 0)
            pltpu.sync_copy(hflat.at[sidx], rows_v)

            def scale16(i, _):
                nvec = nv[pl.ds(i * 16, 16)]
                base = i * 16
                for l in range(16):
                    rows_v[base + l, pl.ds(0, 16)] = (
                        rows_v[base + l, pl.ds(0, 16)] * nvec[l])
                return 0
            lax.fori_loop(0, 64, scale16, 0)
            pltpu.sync_copy(rows_v, acc_sh.at[didx], add=True)
            return 0
        lax.fori_loop(0, nch, chunk, 0)

        plsc.subcore_barrier()
        dump_acc(q)
        return 0
    lax.fori_loop(0, 2, do_pass, 0)


def _agg_stage(srcl, dstl, norml, hflat):
    f = pl.kernel(
        _k4_body,
        out_type=jax.ShapeDtypeStruct((4, NPAD, 16), jnp.float32),
        mesh=_mesh(),
        compiler_params=pltpu.CompilerParams(use_tc_tiling_on_sc=False),
        scratch_types=[
            pltpu.VMEM((1024,), jnp.int32),
            pltpu.VMEM((1024,), jnp.int32),
            pltpu.VMEM((1024,), jnp.float32),
            pltpu.VMEM((1024, 16), jnp.float32),
            pltpu.VMEM((1024, 16), jnp.float32),
            pltpu.VMEM_SHARED((NPAD, 16), jnp.float32),
            pltpu.SemaphoreType.DMA,
        ],
    )
    return f(srcl, dstl, norml, hflat)


def _k5_body(agg_ref, h0_ref, h1_ref, h2_ref, h3_ref, dis_ref, cp_ref,
             b1_ref, w2t_ref, b2_ref, out_ref, acc_ref):
    i = pl.program_id(0)
    d = dis_ref[...]
    inv = d * d
    cvec = cp_ref[0] + cp_ref[1] + inv
    b1 = b1_ref[...]
    a = agg_ref[...]
    rowid = jax.lax.broadcasted_iota(jnp.int32, (BN5, 1), 0) + i * BN5
    valid = rowid < NN
    hq = [h0_ref[...], h1_ref[...], h2_ref[...], h3_ref[...]]
    parts = []
    for q in range(4):
        gq = jnp.maximum(
            a[q] + hq[q] * inv[:, None] + b1[0, 16 * q:16 * q + 16], 0.0)
        gq = jnp.where(valid, gq, 0.0)
        parts.append(
            jnp.dot(cvec[None, :], gq, preferred_element_type=jnp.float32))
    part = jnp.concatenate(parts, axis=1)

    @pl.when(i == 0)
    def _():
        acc_ref[...] = jnp.zeros_like(acc_ref)

    acc_ref[...] += part

    @pl.when(i == pl.num_programs(0) - 1)
    def _():
        out_ref[...] = (
            jnp.dot(acc_ref[...] * (1.0 / NN), w2t_ref[...],
                    preferred_element_type=jnp.float32)
            + b2_ref[...])


def _finish_stage(agg, h0, h1, h2, h3, dis, cp, b1, w2t, b2):
    return pl.pallas_call(
        _k5_body,
        grid=(NPAD // BN5,),
        in_specs=[
            pl.BlockSpec((4, BN5, 16), lambda i: (0, i, 0)),
            pl.BlockSpec((BN5, 16), lambda i: (i, 0)),
            pl.BlockSpec((BN5, 16), lambda i: (i, 0)),
            pl.BlockSpec((BN5, 16), lambda i: (i, 0)),
            pl.BlockSpec((BN5, 16), lambda i: (i, 0)),
            pl.BlockSpec((BN5,), lambda i: (i,)),
            pl.BlockSpec((2, BN5), lambda i: (0, i)),
            pl.BlockSpec((1, 64), lambda i: (0, 0)),
            pl.BlockSpec((64, 64), lambda i: (0, 0)),
            pl.BlockSpec((1, 64), lambda i: (0, 0)),
        ],
        out_specs=pl.BlockSpec((1, 64), lambda i: (0, 0)),
        out_shape=jax.ShapeDtypeStruct((1, 64), jnp.float32),
        scratch_shapes=[pltpu.VMEM((1, 64), jnp.float32)],
    )(agg, h0, h1, h2, h3, dis, cp, b1, w2t, b2)


def _band_matrices(conv1_w, conv2_w):
    w1 = conv1_w[0, 0]
    w2 = conv2_w[0, 0]
    i1 = jnp.arange(TT)[:, None]
    s1 = jnp.arange(S1)[None, :]
    d1 = i1 - 4 * s1
    a1f = jnp.zeros((TT, S1), jnp.float32)
    for k in range(KW):
        a1f = a1f + jnp.where(d1 == 2 * k, w1[k], 0.0)
    i2 = jnp.arange(S1)[:, None]
    t2 = jnp.arange(S2)[None, :]
    d2 = i2 - t2
    a2 = jnp.zeros((S1, S2), jnp.float32)
    for k in range(KW):
        a2 = a2 + jnp.where(d2 == k, w2[k], 0.0)
    return a1f, a2


def kernel(node_features, edge_index, edge_attributes, conv1_w, conv1_b,
           conv2_w, conv2_b, W1, b1, W2, b2):
    src = edge_index[0].astype(jnp.int32)
    dst = edge_index[1].astype(jnp.int32)
    ew = edge_attributes.astype(jnp.float32)
    npadlen = EPAD - E0
    pad_idx = jnp.full((npadlen,), NN - 1, jnp.int32)
    srcl = jnp.concatenate([src, pad_idx])
    dstl = jnp.concatenate([dst, pad_idx])
    ewl = jnp.concatenate([ew, jnp.zeros((npadlen,), jnp.float32)])

    a1f, a2 = _band_matrices(conv1_w, conv2_w)
    sb = jnp.stack([conv1_b[0], conv2_b[0]]).reshape(1, 2)

    h0, h1, h2, h3 = _dense_stage(node_features, a1f, a2, W1.T, sb)
    degp = _deg_stage(dstl, ewl)
    dis = _dis_stage(degp)
    norml, cp = _norm_stage(srcl, dstl, ewl, dis)
    hflat = jnp.concatenate([h0, h1, h2, h3], axis=0)
    agg = _agg_stage(srcl, dstl, norml, hflat)
    out = _finish_stage(agg, h0, h1, h2, h3, dis, cp, b1.reshape(1, 64),
                        W2.T, b2.reshape(1, 64))
    return out


# speedup vs baseline: 19.8210x; 1.2391x over previous
import functools
import jax
import jax.numpy as jnp
import numpy as np
from jax import lax
from jax.experimental import pallas as pl
from jax.experimental.pallas import tpu as pltpu
from jax.experimental.pallas import tpu_sc as plsc

NN = 50000
TT = 518
S1 = 114
S2 = 83
KW = 32
E0 = 800000
EPAD = 819200
EROWS = EPAD // 128
NPAD = 51200
NC, NS = 2, 16
BN = 2000
GRID_N = NN // BN
BN5 = 2048

@functools.lru_cache(maxsize=1)
def _mesh():
    return plsc.VectorSubcoreMesh(
        core_axis_name="c", subcore_axis_name="s",
        num_cores=NC, num_subcores=NS)


def _k1_body(x_ref, a1_ref, a2_ref, w1t_ref, sb_ref, h0_ref, h1_ref,
             h2_ref, h3_ref):
    x = x_ref[...]
    t1 = jnp.maximum(
        jnp.dot(x, a1_ref[...], preferred_element_type=jnp.float32)
        + sb_ref[0, 0], 0.0)
    t2 = jnp.maximum(
        jnp.dot(t1, a2_ref[...], preferred_element_type=jnp.float32)
        + sb_ref[0, 1], 0.0)
    h = jnp.dot(t2, w1t_ref[...], preferred_element_type=jnp.float32)
    h0_ref[...] = h[:, 0:16]
    h1_ref[...] = h[:, 16:32]
    h2_ref[...] = h[:, 32:48]
    h3_ref[...] = h[:, 48:64]


def _dense_stage(nf, a1f, a2, w1t, sb):
    return pl.pallas_call(
        _k1_body,
        grid=(GRID_N,),
        in_specs=[
            pl.BlockSpec((BN, TT), lambda i: (i, 0)),
            pl.BlockSpec((TT, S1), lambda i: (0, 0)),
            pl.BlockSpec((S1, S2), lambda i: (0, 0)),
            pl.BlockSpec((S2, 64), lambda i: (0, 0)),
            pl.BlockSpec((1, 2), lambda i: (0, 0)),
        ],
        out_specs=[
            pl.BlockSpec((BN, 16), lambda i: (i, 0)) for _ in range(4)
        ],
        out_shape=[
            jax.ShapeDtypeStruct((NPAD, 16), jnp.float32) for _ in range(4)
        ],
    )(nf, a1f, a2, w1t, sb)


def _dis_body(dp_ref, dis_ref):
    p = dp_ref[...]
    dis_ref[...] = lax.rsqrt(1.0 + p[0] + p[1])


def _dis_stage(degp):
    return pl.pallas_call(
        _dis_body,
        grid=(1,),
        in_specs=[pl.BlockSpec((2, NPAD), lambda i: (0, 0))],
        out_specs=pl.BlockSpec((NPAD,), lambda i: (0,)),
        out_shape=jax.ShapeDtypeStruct((NPAD,), jnp.float32),
    )(degp)


def _zero_1d(ref, n):
    def body(i, _):
        ref[pl.ds(i * 16, 16)] = jnp.zeros((16,), jnp.float32)
        return 0
    lax.fori_loop(0, n // 16, body, 0)


def _k2_body(dstl, ewl, degp, idx_v, val_v, stripe_v, acc_sh, sem):
    cid = lax.axis_index("c")
    sid = lax.axis_index("s")
    wid = sid * NC + cid
    nbase = pl.multiple_of(sid * (NPAD // NS), 8)

    _zero_1d(stripe_v, NPAD // NS)
    pltpu.sync_copy(stripe_v, acc_sh.at[pl.ds(nbase, NPAD // NS)])
    plsc.subcore_barrier()

    ept = EPAD // (NC * NS)
    ebase = wid * ept

    def chunk(ch, _):
        e0 = ebase + ch * 1024
        pltpu.sync_copy(dstl.at[pl.ds(e0, 1024)], idx_v)
        pltpu.sync_copy(ewl.at[pl.ds(e0, 1024)], val_v)
        pltpu.sync_copy(val_v, acc_sh.at[idx_v], add=True)
        return 0
    lax.fori_loop(0, ept // 1024, chunk, 0)

    plsc.subcore_barrier()
    pltpu.sync_copy(acc_sh.at[pl.ds(nbase, NPAD // NS)], stripe_v)
    pltpu.sync_copy(stripe_v, degp.at[cid, pl.ds(nbase, NPAD // NS)])


def _deg_stage(dstl, ewl):
    f = pl.kernel(
        _k2_body,
        out_type=jax.ShapeDtypeStruct((NC, NPAD), jnp.float32),
        mesh=_mesh(),
        compiler_params=pltpu.CompilerParams(use_tc_tiling_on_sc=False),
        scratch_types=[
            pltpu.VMEM((1024,), jnp.int32),
            pltpu.VMEM((1024,), jnp.float32),
            pltpu.VMEM((NPAD // NS,), jnp.float32),
            pltpu.VMEM_SHARED((NPAD,), jnp.float32),
            pltpu.SemaphoreType.DMA,
        ],
    )
    return f(dstl, ewl)


def _k3_body(srcl, dstl, ewl, dis, norml, cp, sidx, didx, ewv, dsv, ddv, nv,
             stripe_v, acc_sh, sem):
    cid = lax.axis_index("c")
    sid = lax.axis_index("s")
    wid = sid * NC + cid
    nbase = pl.multiple_of(sid * (NPAD // NS), 8)

    _zero_1d(stripe_v, NPAD // NS)
    pltpu.sync_copy(stripe_v, acc_sh.at[pl.ds(nbase, NPAD // NS)])
    plsc.subcore_barrier()

    ept = EPAD // (NC * NS)
    ebase = wid * ept

    def chunk(ch, _):
        e0 = ebase + ch * 1024
        pltpu.sync_copy(srcl.at[pl.ds(e0, 1024)], sidx)
        pltpu.sync_copy(dstl.at[pl.ds(e0, 1024)], didx)
        pltpu.sync_copy(ewl.at[pl.ds(e0, 1024)], ewv)
        pltpu.sync_copy(dis.at[sidx], dsv)
        pltpu.sync_copy(dis.at[didx], ddv)

        def mul16(i, _):
            sl = pl.ds(i * 16, 16)
            nv[sl] = dsv[sl] * ewv[sl] * ddv[sl]
            return 0
        lax.fori_loop(0, 64, mul16, 0)
        pltpu.sync_copy(nv, norml.at[pl.ds(e0, 1024)])
        pltpu.sync_copy(nv, acc_sh.at[sidx], add=True)
        return 0
    lax.fori_loop(0, ept // 1024, chunk, 0)

    plsc.subcore_barrier()
    pltpu.sync_copy(acc_sh.at[pl.ds(nbase, NPAD // NS)], stripe_v)
    pltpu.sync_copy(stripe_v, cp.at[cid, pl.ds(nbase, NPAD // NS)])


def _norm_stage(srcl, dstl, ewl, dis):
    f = pl.kernel(
        _k3_body,
        out_type=(
            jax.ShapeDtypeStruct((EPAD,), jnp.float32),
            jax.ShapeDtypeStruct((NC, NPAD), jnp.float32),
        ),
        mesh=_mesh(),
        compiler_params=pltpu.CompilerParams(use_tc_tiling_on_sc=False),
        scratch_types=[
            pltpu.VMEM((1024,), jnp.int32),
            pltpu.VMEM((1024,), jnp.int32),
            pltpu.VMEM((1024,), jnp.float32),
            pltpu.VMEM((1024,), jnp.float32),
            pltpu.VMEM((1024,), jnp.float32),
            pltpu.VMEM((1024,), jnp.float32),
            pltpu.VMEM((NPAD // NS,), jnp.float32),
            pltpu.VMEM_SHARED((NPAD,), jnp.float32),
            pltpu.SemaphoreType.DMA,
        ],
    )
    return f(srcl, dstl, ewl, dis)


def _k4_body(srcl, dstl, norml, hflat, agg,
             sidx0, sidx1, sidx2, didx0, didx1, didx2, nv0, nv1, nv2,
             rows0, rows1, rows2, zb, acc_sh,
             sl0, sl1, sl2, sg0, sg1, sg2, ss0, ss1, ss2):
    cid = lax.axis_index("c")
    sid = lax.axis_index("s")
    nbase = pl.multiple_of(sid * (NPAD // NS), 8)

    SIDX = (sidx0, sidx1, sidx2)
    DIDX = (didx0, didx1, didx2)
    NV = (nv0, nv1, nv2)
    ROWS = (rows0, rows1, rows2)
    SL = (sl0, sl1, sl2)
    SG = (sg0, sg1, sg2)
    SS = (ss0, ss1, ss2)

    def zb_zero(i, _):
        zb[i, pl.ds(0, 16)] = jnp.zeros((16,), jnp.float32)
        return 0

    def zero_acc():
        lax.fori_loop(0, 1024, zb_zero, 0)
        for t in range(3):
            pltpu.sync_copy(zb, acc_sh.at[pl.ds(nbase + t * 1024, 1024)])
        pltpu.sync_copy(zb.at[pl.ds(0, 128)],
                        acc_sh.at[pl.ds(nbase + 3072, 128)])

    def dump_acc(q):
        for t in range(3):
            pltpu.sync_copy(acc_sh.at[pl.ds(nbase + t * 1024, 1024)], zb)
            pltpu.sync_copy(zb, agg.at[q, pl.ds(nbase + t * 1024, 1024)])
        pltpu.sync_copy(acc_sh.at[pl.ds(nbase + 3072, 128)],
                        zb.at[pl.ds(0, 128)])
        pltpu.sync_copy(zb.at[pl.ds(0, 128)],
                        agg.at[q, pl.ds(nbase + 3072, 128)])

    edges_per_tile = EPAD // NS
    ebase = sid * edges_per_tile
    nch = edges_per_tile // 1024

    def issue_lin(ch, b):
        e0 = ebase + ch * 1024
        pltpu.async_copy(srcl.at[pl.ds(e0, 1024)], SIDX[b], SL[b])
        pltpu.async_copy(dstl.at[pl.ds(e0, 1024)], DIDX[b], SL[b])
        pltpu.async_copy(norml.at[pl.ds(e0, 1024)], NV[b], SL[b])

    def wait_lin(b):
        pltpu.make_async_copy(srcl.at[pl.ds(0, 1024)], SIDX[b], SL[b]).wait()
        pltpu.make_async_copy(dstl.at[pl.ds(0, 1024)], DIDX[b], SL[b]).wait()
        pltpu.make_async_copy(norml.at[pl.ds(0, 1024)], NV[b], SL[b]).wait()

    def addq(b, qoff):
        def f(i, _):
            sl = pl.ds(i * 16, 16)
            SIDX[b][sl] = SIDX[b][sl] + qoff
            return 0
        lax.fori_loop(0, 64, f, 0)

    def issue_gat(b):
        pltpu.async_copy(hflat.at[SIDX[b]], ROWS[b], SG[b])

    def wait_gat(b):
        pltpu.make_async_copy(hflat.at[SIDX[b]], ROWS[b], SG[b]).wait()

    def issue_sca(b):
        pltpu.async_copy(ROWS[b], acc_sh.at[DIDX[b]], SS[b], add=True)

    def wait_sca(b):
        pltpu.make_async_copy(ROWS[b], acc_sh.at[DIDX[b]], SS[b]).wait()

    def scale(b):
        def f(i, _):
            nvec = NV[b][pl.ds(i * 16, 16)]
            base = i * 16
            for l in range(16):
                ROWS[b][base + l, pl.ds(0, 16)] = (
                    ROWS[b][base + l, pl.ds(0, 16)] * nvec[l])
            return 0
        lax.fori_loop(0, 64, f, 0)

    def body(ch, b, qoff, first=False, has1=True, has2=True):
        nb = (b + 1) % 3
        pb = (b + 2) % 3
        if not first:
            wait_sca(pb)
        if has2:
            issue_lin(ch + 2, pb)
        wait_gat(b)
        if has1:
            wait_lin(nb)
            addq(nb, qoff)
            issue_gat(nb)
        scale(b)
        issue_sca(b)

    def do_pass(p, _):
        q = cid * 2 + p
        qoff = q * NPAD
        zero_acc()
        plsc.subcore_barrier()

        issue_lin(0, 0)
        issue_lin(1, 1)
        wait_lin(0)
        addq(0, qoff)
        issue_gat(0)
        body(0, 0, qoff, first=True)

        def steady(pp, _):
            ch = 3 * pp + 1
            body(ch, 1, qoff)
            body(ch + 1, 2, qoff)
            body(ch + 2, 0, qoff)
            return 0
        lax.fori_loop(0, (nch - 2) // 3, steady, 0)

        body(nch - 1, (nch - 1) % 3, qoff, has1=False, has2=False)
        wait_sca((nch - 1) % 3)

        plsc.subcore_barrier()
        dump_acc(q)
        return 0
    lax.fori_loop(0, 2, do_pass, 0)


def _agg_stage(srcl, dstl, norml, hflat):
    f = pl.kernel(
        _k4_body,
        out_type=jax.ShapeDtypeStruct((4, NPAD, 16), jnp.float32),
        mesh=_mesh(),
        compiler_params=pltpu.CompilerParams(use_tc_tiling_on_sc=False),
        scratch_types=(
            [pltpu.VMEM((1024,), jnp.int32) for _ in range(6)]
            + [pltpu.VMEM((1024,), jnp.float32) for _ in range(3)]
            + [pltpu.VMEM((1024, 16), jnp.float32) for _ in range(4)]
            + [pltpu.VMEM_SHARED((NPAD, 16), jnp.float32)]
            + [pltpu.SemaphoreType.DMA for _ in range(9)]
        ),
    )
    return f(srcl, dstl, norml, hflat)


def _k5_body(agg_ref, h0_ref, h1_ref, h2_ref, h3_ref, dis_ref, cp_ref,
             b1_ref, w2t_ref, b2_ref, out_ref, acc_ref):
    i = pl.program_id(0)
    d = dis_ref[...]
    inv = d * d
    cvec = cp_ref[0] + cp_ref[1] + inv
    b1 = b1_ref[...]
    a = agg_ref[...]
    rowid = jax.lax.broadcasted_iota(jnp.int32, (BN5, 1), 0) + i * BN5
    valid = rowid < NN
    hq = [h0_ref[...], h1_ref[...], h2_ref[...], h3_ref[...]]
    parts = []
    for q in range(4):
        gq = jnp.maximum(
            a[q] + hq[q] * inv[:, None] + b1[0, 16 * q:16 * q + 16], 0.0)
        gq = jnp.where(valid, gq, 0.0)
        parts.append(
            jnp.dot(cvec[None, :], gq, preferred_element_type=jnp.float32))
    part = jnp.concatenate(parts, axis=1)

    @pl.when(i == 0)
    def _():
        acc_ref[...] = jnp.zeros_like(acc_ref)

    acc_ref[...] += part

    @pl.when(i == pl.num_programs(0) - 1)
    def _():
        out_ref[...] = (
            jnp.dot(acc_ref[...] * (1.0 / NN), w2t_ref[...],
                    preferred_element_type=jnp.float32)
            + b2_ref[...])


def _finish_stage(agg, h0, h1, h2, h3, dis, cp, b1, w2t, b2):
    return pl.pallas_call(
        _k5_body,
        grid=(NPAD // BN5,),
        in_specs=[
            pl.BlockSpec((4, BN5, 16), lambda i: (0, i, 0)),
            pl.BlockSpec((BN5, 16), lambda i: (i, 0)),
            pl.BlockSpec((BN5, 16), lambda i: (i, 0)),
            pl.BlockSpec((BN5, 16), lambda i: (i, 0)),
            pl.BlockSpec((BN5, 16), lambda i: (i, 0)),
            pl.BlockSpec((BN5,), lambda i: (i,)),
            pl.BlockSpec((2, BN5), lambda i: (0, i)),
            pl.BlockSpec((1, 64), lambda i: (0, 0)),
            pl.BlockSpec((64, 64), lambda i: (0, 0)),
            pl.BlockSpec((1, 64), lambda i: (0, 0)),
        ],
        out_specs=pl.BlockSpec((1, 64), lambda i: (0, 0)),
        out_shape=jax.ShapeDtypeStruct((1, 64), jnp.float32),
        scratch_shapes=[pltpu.VMEM((1, 64), jnp.float32)],
    )(agg, h0, h1, h2, h3, dis, cp, b1, w2t, b2)


def _band_matrices(conv1_w, conv2_w):
    w1 = conv1_w[0, 0]
    w2 = conv2_w[0, 0]
    i1 = jnp.arange(TT)[:, None]
    s1 = jnp.arange(S1)[None, :]
    d1 = i1 - 4 * s1
    a1f = jnp.zeros((TT, S1), jnp.float32)
    for k in range(KW):
        a1f = a1f + jnp.where(d1 == 2 * k, w1[k], 0.0)
    i2 = jnp.arange(S1)[:, None]
    t2 = jnp.arange(S2)[None, :]
    d2 = i2 - t2
    a2 = jnp.zeros((S1, S2), jnp.float32)
    for k in range(KW):
        a2 = a2 + jnp.where(d2 == k, w2[k], 0.0)
    return a1f, a2


def kernel(node_features, edge_index, edge_attributes, conv1_w, conv1_b,
           conv2_w, conv2_b, W1, b1, W2, b2):
    src = edge_index[0].astype(jnp.int32)
    dst = edge_index[1].astype(jnp.int32)
    ew = edge_attributes.astype(jnp.float32)
    npadlen = EPAD - E0
    pad_idx = jnp.full((npadlen,), NN - 1, jnp.int32)
    srcl = jnp.concatenate([src, pad_idx])
    dstl = jnp.concatenate([dst, pad_idx])
    ewl = jnp.concatenate([ew, jnp.zeros((npadlen,), jnp.float32)])

    a1f, a2 = _band_matrices(conv1_w, conv2_w)
    sb = jnp.stack([conv1_b[0], conv2_b[0]]).reshape(1, 2)

    h0, h1, h2, h3 = _dense_stage(node_features, a1f, a2, W1.T, sb)
    degp = _deg_stage(dstl, ewl)
    dis = _dis_stage(degp)
    norml, cp = _norm_stage(srcl, dstl, ewl, dis)
    hflat = jnp.concatenate([h0, h1, h2, h3], axis=0)
    agg = _agg_stage(srcl, dstl, norml, hflat)
    out = _finish_stage(agg, h0, h1, h2, h3, dis, cp, b1.reshape(1, 64),
                        W2.T, b2.reshape(1, 64))
    return out


# speedup vs baseline: 20.7667x; 1.0477x over previous
import functools
import jax
import jax.numpy as jnp
import numpy as np
from jax import lax
from jax.experimental import pallas as pl
from jax.experimental.pallas import tpu as pltpu
from jax.experimental.pallas import tpu_sc as plsc

NN = 50000
TT = 518
S1 = 114
S2 = 83
KW = 32
E0 = 800000
EPAD = 819200
EROWS = EPAD // 128
NPAD = 51200
NC, NS = 2, 16
BN = 2000
GRID_N = NN // BN
BN5 = 2048

@functools.lru_cache(maxsize=1)
def _mesh():
    return plsc.VectorSubcoreMesh(
        core_axis_name="c", subcore_axis_name="s",
        num_cores=NC, num_subcores=NS)


def _k1_body(x_ref, a1_ref, a2_ref, w1t_ref, sb_ref, h0_ref, h1_ref,
             h2_ref, h3_ref):
    x = x_ref[...]
    t1 = jnp.maximum(
        jnp.dot(x, a1_ref[...], preferred_element_type=jnp.float32)
        + sb_ref[0, 0], 0.0)
    t2 = jnp.maximum(
        jnp.dot(t1, a2_ref[...], preferred_element_type=jnp.float32)
        + sb_ref[0, 1], 0.0)
    h = jnp.dot(t2, w1t_ref[...], preferred_element_type=jnp.float32)
    h0_ref[...] = h[:, 0:16]
    h1_ref[...] = h[:, 16:32]
    h2_ref[...] = h[:, 32:48]
    h3_ref[...] = h[:, 48:64]


def _dense_stage(nf, a1f, a2, w1t, sb):
    return pl.pallas_call(
        _k1_body,
        grid=(GRID_N,),
        in_specs=[
            pl.BlockSpec((BN, TT), lambda i: (i, 0)),
            pl.BlockSpec((TT, S1), lambda i: (0, 0)),
            pl.BlockSpec((S1, S2), lambda i: (0, 0)),
            pl.BlockSpec((S2, 64), lambda i: (0, 0)),
            pl.BlockSpec((1, 2), lambda i: (0, 0)),
        ],
        out_specs=[
            pl.BlockSpec((BN, 16), lambda i: (i, 0)) for _ in range(4)
        ],
        out_shape=[
            jax.ShapeDtypeStruct((NPAD, 16), jnp.float32) for _ in range(4)
        ],
    )(nf, a1f, a2, w1t, sb)


def _dis_body(dp_ref, dis_ref):
    p = dp_ref[...]
    dis_ref[...] = lax.rsqrt(1.0 + p[0] + p[1])


def _dis_stage(degp):
    return pl.pallas_call(
        _dis_body,
        grid=(1,),
        in_specs=[pl.BlockSpec((2, NPAD), lambda i: (0, 0))],
        out_specs=pl.BlockSpec((NPAD,), lambda i: (0,)),
        out_shape=jax.ShapeDtypeStruct((NPAD,), jnp.float32),
    )(degp)


def _zero_1d(ref, n):
    def body(i, _):
        ref[pl.ds(i * 16, 16)] = jnp.zeros((16,), jnp.float32)
        return 0
    lax.fori_loop(0, n // 16, body, 0)


def _k2_body(dstl, ewl, degp, idx_v, val_v, stripe_v, acc_sh, sem):
    cid = lax.axis_index("c")
    sid = lax.axis_index("s")
    wid = sid * NC + cid
    nbase = pl.multiple_of(sid * (NPAD // NS), 8)

    _zero_1d(stripe_v, NPAD // NS)
    pltpu.sync_copy(stripe_v, acc_sh.at[pl.ds(nbase, NPAD // NS)])
    plsc.subcore_barrier()

    ept = EPAD // (NC * NS)
    ebase = wid * ept

    def chunk(ch, _):
        e0 = ebase + ch * 1024
        pltpu.sync_copy(dstl.at[pl.ds(e0, 1024)], idx_v)
        pltpu.sync_copy(ewl.at[pl.ds(e0, 1024)], val_v)
        pltpu.sync_copy(val_v, acc_sh.at[idx_v], add=True)
        return 0
    lax.fori_loop(0, ept // 1024, chunk, 0)

    plsc.subcore_barrier()
    pltpu.sync_copy(acc_sh.at[pl.ds(nbase, NPAD // NS)], stripe_v)
    pltpu.sync_copy(stripe_v, degp.at[cid, pl.ds(nbase, NPAD // NS)])


def _deg_stage(dstl, ewl):
    f = pl.kernel(
        _k2_body,
        out_type=jax.ShapeDtypeStruct((NC, NPAD), jnp.float32),
        mesh=_mesh(),
        compiler_params=pltpu.CompilerParams(use_tc_tiling_on_sc=False),
        scratch_types=[
            pltpu.VMEM((1024,), jnp.int32),
            pltpu.VMEM((1024,), jnp.float32),
            pltpu.VMEM((NPAD // NS,), jnp.float32),
            pltpu.VMEM_SHARED((NPAD,), jnp.float32),
            pltpu.SemaphoreType.DMA,
        ],
    )
    return f(dstl, ewl)


def _k3_body(srcl, dstl, ewl, dis, norml, cp,
             sidx0, sidx1, sidx2, didx0, didx1, didx2, ewv0, ewv1, ewv2,
             dsv0, dsv1, dsv2, ddv0, ddv1, ddv2, nv0, nv1, nv2,
             stripe_v, acc_sh,
             sl0, sl1, sl2, sg0, sg1, sg2, ss0, ss1, ss2, sw0, sw1, sw2):
    cid = lax.axis_index("c")
    sid = lax.axis_index("s")
    wid = sid * NC + cid
    nbase = pl.multiple_of(sid * (NPAD // NS), 8)

    SIDX = (sidx0, sidx1, sidx2)
    DIDX = (didx0, didx1, didx2)
    EWV = (ewv0, ewv1, ewv2)
    DSV = (dsv0, dsv1, dsv2)
    DDV = (ddv0, ddv1, ddv2)
    NV = (nv0, nv1, nv2)
    SL = (sl0, sl1, sl2)
    SG = (sg0, sg1, sg2)
    SS = (ss0, ss1, ss2)
    SW = (sw0, sw1, sw2)

    _zero_1d(stripe_v, NPAD // NS)
    pltpu.sync_copy(stripe_v, acc_sh.at[pl.ds(nbase, NPAD // NS)])
    plsc.subcore_barrier()

    ept = EPAD // (NC * NS)
    ebase = wid * ept
    nch = ept // 1024

    def issue_lin(ch, b):
        e0 = ebase + ch * 1024
        pltpu.async_copy(srcl.at[pl.ds(e0, 1024)], SIDX[b], SL[b])
        pltpu.async_copy(dstl.at[pl.ds(e0, 1024)], DIDX[b], SL[b])
        pltpu.async_copy(ewl.at[pl.ds(e0, 1024)], EWV[b], SL[b])

    def wait_lin(b):
        pltpu.make_async_copy(srcl.at[pl.ds(0, 1024)], SIDX[b], SL[b]).wait()
        pltpu.make_async_copy(dstl.at[pl.ds(0, 1024)], DIDX[b], SL[b]).wait()
        pltpu.make_async_copy(ewl.at[pl.ds(0, 1024)], EWV[b], SL[b]).wait()

    def issue_gat(b):
        pltpu.async_copy(dis.at[SIDX[b]], DSV[b], SG[b])
        pltpu.async_copy(dis.at[DIDX[b]], DDV[b], SG[b])

    def wait_gat(b):
        pltpu.make_async_copy(dis.at[SIDX[b]], DSV[b], SG[b]).wait()
        pltpu.make_async_copy(dis.at[DIDX[b]], DDV[b], SG[b]).wait()

    def issue_out(ch, b):
        e0 = ebase + ch * 1024
        pltpu.async_copy(NV[b], norml.at[pl.ds(e0, 1024)], SW[b])
        pltpu.async_copy(NV[b], acc_sh.at[SIDX[b]], SS[b], add=True)

    def wait_out(b):
        pltpu.make_async_copy(NV[b], norml.at[pl.ds(0, 1024)], SW[b]).wait()
        pltpu.make_async_copy(NV[b], acc_sh.at[SIDX[b]], SS[b]).wait()

    def compute(b):
        def f(i, _):
            sl = pl.ds(i * 16, 16)
            NV[b][sl] = DSV[b][sl] * EWV[b][sl] * DDV[b][sl]
            return 0
        lax.fori_loop(0, 64, f, 0)

    def body(ch, b, first=False, has1=True, has2=True):
        nb = (b + 1) % 3
        pb = (b + 2) % 3
        if not first:
            wait_out(pb)
        if has2:
            issue_lin(ch + 2, pb)
        wait_gat(b)
        if has1:
            wait_lin(nb)
            issue_gat(nb)
        compute(b)
        issue_out(ch, b)

    issue_lin(0, 0)
    issue_lin(1, 1)
    wait_lin(0)
    issue_gat(0)
    body(0, 0, first=True)

    def steady(pp, _):
        ch = 3 * pp + 1
        body(ch, 1)
        body(ch + 1, 2)
        body(ch + 2, 0)
        return 0
    lax.fori_loop(0, (nch - 4) // 3, steady, 0)

    body(nch - 3, (nch - 3) % 3)
    body(nch - 2, (nch - 2) % 3, has2=False)
    body(nch - 1, (nch - 1) % 3, has1=False, has2=False)
    wait_out((nch - 1) % 3)

    plsc.subcore_barrier()
    pltpu.sync_copy(acc_sh.at[pl.ds(nbase, NPAD // NS)], stripe_v)
    pltpu.sync_copy(stripe_v, cp.at[cid, pl.ds(nbase, NPAD // NS)])


def _norm_stage(srcl, dstl, ewl, dis):
    f = pl.kernel(
        _k3_body,
        out_type=(
            jax.ShapeDtypeStruct((EPAD,), jnp.float32),
            jax.ShapeDtypeStruct((NC, NPAD), jnp.float32),
        ),
        mesh=_mesh(),
        compiler_params=pltpu.CompilerParams(use_tc_tiling_on_sc=False),
        scratch_types=(
            [pltpu.VMEM((1024,), jnp.int32) for _ in range(6)]
            + [pltpu.VMEM((1024,), jnp.float32) for _ in range(12)]
            + [pltpu.VMEM((NPAD // NS,), jnp.float32)]
            + [pltpu.VMEM_SHARED((NPAD,), jnp.float32)]
            + [pltpu.SemaphoreType.DMA for _ in range(12)]
        ),
    )
    return f(srcl, dstl, ewl, dis)


def _k4_body(srcl, dstl, norml, hflat, agg,
             sidx0, sidx1, sidx2, didx0, didx1, didx2, nv0, nv1, nv2,
             rows0, rows1, rows2, zb, acc_sh,
             sl0, sl1, sl2, sg0, sg1, sg2, ss0, ss1, ss2):
    cid = lax.axis_index("c")
    sid = lax.axis_index("s")
    nbase = pl.multiple_of(sid * (NPAD // NS), 8)

    SIDX = (sidx0, sidx1, sidx2)
    DIDX = (didx0, didx1, didx2)
    NV = (nv0, nv1, nv2)
    ROWS = (rows0, rows1, rows2)
    SL = (sl0, sl1, sl2)
    SG = (sg0, sg1, sg2)
    SS = (ss0, ss1, ss2)

    def zb_zero(i, _):
        zb[i, pl.ds(0, 16)] = jnp.zeros((16,), jnp.float32)
        return 0

    def zero_acc():
        lax.fori_loop(0, 1024, zb_zero, 0)
        for t in range(3):
            pltpu.sync_copy(zb, acc_sh.at[pl.ds(nbase + t * 1024, 1024)])
        pltpu.sync_copy(zb.at[pl.ds(0, 128)],
                        acc_sh.at[pl.ds(nbase + 3072, 128)])

    def dump_acc(q):
        for t in range(3):
            pltpu.sync_copy(acc_sh.at[pl.ds(nbase + t * 1024, 1024)], zb)
            pltpu.sync_copy(zb, agg.at[q, pl.ds(nbase + t * 1024, 1024)])
        pltpu.sync_copy(acc_sh.at[pl.ds(nbase + 3072, 128)],
                        zb.at[pl.ds(0, 128)])
        pltpu.sync_copy(zb.at[pl.ds(0, 128)],
                        agg.at[q, pl.ds(nbase + 3072, 128)])

    edges_per_tile = EPAD // NS
    ebase = sid * edges_per_tile
    nch = edges_per_tile // 1024

    def issue_lin(ch, b):
        e0 = ebase + ch * 1024
        pltpu.async_copy(srcl.at[pl.ds(e0, 1024)], SIDX[b], SL[b])
        pltpu.async_copy(dstl.at[pl.ds(e0, 1024)], DIDX[b], SL[b])
        pltpu.async_copy(norml.at[pl.ds(e0, 1024)], NV[b], SL[b])

    def wait_lin(b):
        pltpu.make_async_copy(srcl.at[pl.ds(0, 1024)], SIDX[b], SL[b]).wait()
        pltpu.make_async_copy(dstl.at[pl.ds(0, 1024)], DIDX[b], SL[b]).wait()
        pltpu.make_async_copy(norml.at[pl.ds(0, 1024)], NV[b], SL[b]).wait()

    def addq(b, qoff):
        def f(i, _):
            sl = pl.ds(i * 16, 16)
            SIDX[b][sl] = SIDX[b][sl] + qoff
            return 0
        lax.fori_loop(0, 64, f, 0)

    def issue_gat(b):
        pltpu.async_copy(hflat.at[SIDX[b]], ROWS[b], SG[b])

    def wait_gat(b):
        pltpu.make_async_copy(hflat.at[SIDX[b]], ROWS[b], SG[b]).wait()

    def issue_sca(b):
        pltpu.async_copy(ROWS[b], acc_sh.at[DIDX[b]], SS[b], add=True)

    def wait_sca(b):
        pltpu.make_async_copy(ROWS[b], acc_sh.at[DIDX[b]], SS[b]).wait()

    def scale(b):
        def f(i, _):
            nvec = NV[b][pl.ds(i * 16, 16)]
            base = i * 16
            for l in range(16):
                ROWS[b][base + l, pl.ds(0, 16)] = (
                    ROWS[b][base + l, pl.ds(0, 16)] * nvec[l])
            return 0
        lax.fori_loop(0, 64, f, 0)

    def body(ch, b, qoff, first=False, has1=True, has2=True):
        nb = (b + 1) % 3
        pb = (b + 2) % 3
        if not first:
            wait_sca(pb)
        if has2:
            issue_lin(ch + 2, pb)
        wait_gat(b)
        if has1:
            wait_lin(nb)
            addq(nb, qoff)
            issue_gat(nb)
        scale(b)
        issue_sca(b)

    def do_pass(p, _):
        q = cid * 2 + p
        qoff = q * NPAD
        zero_acc()
        plsc.subcore_barrier()

        issue_lin(0, 0)
        issue_lin(1, 1)
        wait_lin(0)
        addq(0, qoff)
        issue_gat(0)
        body(0, 0, qoff, first=True)

        def steady(pp, _):
            ch = 3 * pp + 1
            body(ch, 1, qoff)
            body(ch + 1, 2, qoff)
            body(ch + 2, 0, qoff)
            return 0
        lax.fori_loop(0, (nch - 2) // 3, steady, 0)

        body(nch - 1, (nch - 1) % 3, qoff, has1=False, has2=False)
        wait_sca((nch - 1) % 3)

        plsc.subcore_barrier()
        dump_acc(q)
        return 0
    lax.fori_loop(0, 2, do_pass, 0)


def _agg_stage(srcl, dstl, norml, hflat):
    f = pl.kernel(
        _k4_body,
        out_type=jax.ShapeDtypeStruct((4, NPAD, 16), jnp.float32),
        mesh=_mesh(),
        compiler_params=pltpu.CompilerParams(use_tc_tiling_on_sc=False),
        scratch_types=(
            [pltpu.VMEM((1024,), jnp.int32) for _ in range(6)]
            + [pltpu.VMEM((1024,), jnp.float32) for _ in range(3)]
            + [pltpu.VMEM((1024, 16), jnp.float32) for _ in range(4)]
            + [pltpu.VMEM_SHARED((NPAD, 16), jnp.float32)]
            + [pltpu.SemaphoreType.DMA for _ in range(9)]
        ),
    )
    return f(srcl, dstl, norml, hflat)


def _k5_body(agg_ref, h0_ref, h1_ref, h2_ref, h3_ref, dis_ref, cp_ref,
             b1_ref, w2t_ref, b2_ref, out_ref, acc_ref):
    i = pl.program_id(0)
    d = dis_ref[...]
    inv = d * d
    cvec = cp_ref[0] + cp_ref[1] + inv
    b1 = b1_ref[...]
    a = agg_ref[...]
    rowid = jax.lax.broadcasted_iota(jnp.int32, (BN5, 1), 0) + i * BN5
    valid = rowid < NN
    hq = [h0_ref[...], h1_ref[...], h2_ref[...], h3_ref[...]]
    parts = []
    for q in range(4):
        gq = jnp.maximum(
            a[q] + hq[q] * inv[:, None] + b1[0, 16 * q:16 * q + 16], 0.0)
        gq = jnp.where(valid, gq, 0.0)
        parts.append(
            jnp.dot(cvec[None, :], gq, preferred_element_type=jnp.float32))
    part = jnp.concatenate(parts, axis=1)

    @pl.when(i == 0)
    def _():
        acc_ref[...] = jnp.zeros_like(acc_ref)

    acc_ref[...] += part

    @pl.when(i == pl.num_programs(0) - 1)
    def _():
        out_ref[...] = (
            jnp.dot(acc_ref[...] * (1.0 / NN), w2t_ref[...],
                    preferred_element_type=jnp.float32)
            + b2_ref[...])


def _finish_stage(agg, h0, h1, h2, h3, dis, cp, b1, w2t, b2):
    return pl.pallas_call(
        _k5_body,
        grid=(NPAD // BN5,),
        in_specs=[
            pl.BlockSpec((4, BN5, 16), lambda i: (0, i, 0)),
            pl.BlockSpec((BN5, 16), lambda i: (i, 0)),
            pl.BlockSpec((BN5, 16), lambda i: (i, 0)),
            pl.BlockSpec((BN5, 16), lambda i: (i, 0)),
            pl.BlockSpec((BN5, 16), lambda i: (i, 0)),
            pl.BlockSpec((BN5,), lambda i: (i,)),
            pl.BlockSpec((2, BN5), lambda i: (0, i)),
            pl.BlockSpec((1, 64), lambda i: (0, 0)),
            pl.BlockSpec((64, 64), lambda i: (0, 0)),
            pl.BlockSpec((1, 64), lambda i: (0, 0)),
        ],
        out_specs=pl.BlockSpec((1, 64), lambda i: (0, 0)),
        out_shape=jax.ShapeDtypeStruct((1, 64), jnp.float32),
        scratch_shapes=[pltpu.VMEM((1, 64), jnp.float32)],
    )(agg, h0, h1, h2, h3, dis, cp, b1, w2t, b2)


def _band_matrices(conv1_w, conv2_w):
    w1 = conv1_w[0, 0]
    w2 = conv2_w[0, 0]
    i1 = jnp.arange(TT)[:, None]
    s1 = jnp.arange(S1)[None, :]
    d1 = i1 - 4 * s1
    a1f = jnp.zeros((TT, S1), jnp.float32)
    for k in range(KW):
        a1f = a1f + jnp.where(d1 == 2 * k, w1[k], 0.0)
    i2 = jnp.arange(S1)[:, None]
    t2 = jnp.arange(S2)[None, :]
    d2 = i2 - t2
    a2 = jnp.zeros((S1, S2), jnp.float32)
    for k in range(KW):
        a2 = a2 + jnp.where(d2 == k, w2[k], 0.0)
    return a1f, a2


def kernel(node_features, edge_index, edge_attributes, conv1_w, conv1_b,
           conv2_w, conv2_b, W1, b1, W2, b2):
    src = edge_index[0].astype(jnp.int32)
    dst = edge_index[1].astype(jnp.int32)
    ew = edge_attributes.astype(jnp.float32)
    npadlen = EPAD - E0
    pad_idx = jnp.full((npadlen,), NN - 1, jnp.int32)
    srcl = jnp.concatenate([src, pad_idx])
    dstl = jnp.concatenate([dst, pad_idx])
    ewl = jnp.concatenate([ew, jnp.zeros((npadlen,), jnp.float32)])

    a1f, a2 = _band_matrices(conv1_w, conv2_w)
    sb = jnp.stack([conv1_b[0], conv2_b[0]]).reshape(1, 2)

    h0, h1, h2, h3 = _dense_stage(node_features, a1f, a2, W1.T, sb)
    degp = _deg_stage(dstl, ewl)
    dis = _dis_stage(degp)
    norml, cp = _norm_stage(srcl, dstl, ewl, dis)
    hflat = jnp.concatenate([h0, h1, h2, h3], axis=0)
    agg = _agg_stage(srcl, dstl, norml, hflat)
    out = _finish_stage(agg, h0, h1, h2, h3, dis, cp, b1.reshape(1, 64),
                        W2.T, b2.reshape(1, 64))
    return out


# speedup vs baseline: 22.5294x; 1.0849x over previous
import functools
import jax
import jax.numpy as jnp
import numpy as np
from jax import lax
from jax.experimental import pallas as pl
from jax.experimental.pallas import tpu as pltpu
from jax.experimental.pallas import tpu_sc as plsc

NN = 50000
TT = 518
S1 = 114
S2 = 83
KW = 32
E0 = 800000
EPAD = 819200
EROWS = EPAD // 128
NPAD = 51200
NC, NS = 2, 16
BN = 2000
GRID_N = NN // BN
BN5 = 2048

@functools.lru_cache(maxsize=1)
def _mesh():
    return plsc.VectorSubcoreMesh(
        core_axis_name="c", subcore_axis_name="s",
        num_cores=NC, num_subcores=NS)


def _k1_body(x_ref, a1_ref, a2_ref, w1t_ref, sb_ref, h4_ref):
    x = x_ref[...]
    t1 = jnp.maximum(
        jnp.dot(x, a1_ref[...], preferred_element_type=jnp.float32)
        + sb_ref[0, 0], 0.0)
    t2 = jnp.maximum(
        jnp.dot(t1, a2_ref[...], preferred_element_type=jnp.float32)
        + sb_ref[0, 1], 0.0)
    h = jnp.dot(t2, w1t_ref[...], preferred_element_type=jnp.float32)
    h4_ref[...] = jnp.stack(
        [h[:, 0:16], h[:, 16:32], h[:, 32:48], h[:, 48:64]], axis=0)


def _dense_stage(nf, a1f, a2, w1t, sb):
    return pl.pallas_call(
        _k1_body,
        grid=(GRID_N,),
        in_specs=[
            pl.BlockSpec((BN, TT), lambda i: (i, 0)),
            pl.BlockSpec((TT, S1), lambda i: (0, 0)),
            pl.BlockSpec((S1, S2), lambda i: (0, 0)),
            pl.BlockSpec((S2, 64), lambda i: (0, 0)),
            pl.BlockSpec((1, 2), lambda i: (0, 0)),
        ],
        out_specs=pl.BlockSpec((4, BN, 16), lambda i: (0, i, 0)),
        out_shape=jax.ShapeDtypeStruct((4, NPAD, 16), jnp.float32),
    )(nf, a1f, a2, w1t, sb)


def _dis_body(dp_ref, dis_ref):
    p = dp_ref[...]
    dis_ref[...] = lax.rsqrt(1.0 + p[0] + p[1])


def _dis_stage(degp):
    return pl.pallas_call(
        _dis_body,
        grid=(1,),
        in_specs=[pl.BlockSpec((2, NPAD), lambda i: (0, 0))],
        out_specs=pl.BlockSpec((NPAD,), lambda i: (0,)),
        out_shape=jax.ShapeDtypeStruct((NPAD,), jnp.float32),
    )(degp)


def _zero_1d(ref, n):
    def body(i, _):
        ref[pl.ds(i * 16, 16)] = jnp.zeros((16,), jnp.float32)
        return 0
    lax.fori_loop(0, n // 16, body, 0)


def _k2_body(dstl, ewl, degp, idx_v, val_v, stripe_v, acc_sh, sem):
    cid = lax.axis_index("c")
    sid = lax.axis_index("s")
    wid = sid * NC + cid
    nbase = pl.multiple_of(sid * (NPAD // NS), 8)

    _zero_1d(stripe_v, NPAD // NS)
    pltpu.sync_copy(stripe_v, acc_sh.at[pl.ds(nbase, NPAD // NS)])
    plsc.subcore_barrier()

    ept = EPAD // (NC * NS)
    ebase = wid * ept

    def chunk(ch, _):
        e0 = ebase + ch * 1024
        pltpu.sync_copy(dstl.at[pl.ds(e0, 1024)], idx_v)
        pltpu.sync_copy(ewl.at[pl.ds(e0, 1024)], val_v)
        pltpu.sync_copy(val_v, acc_sh.at[idx_v], add=True)
        return 0
    lax.fori_loop(0, ept // 1024, chunk, 0)

    plsc.subcore_barrier()
    pltpu.sync_copy(acc_sh.at[pl.ds(nbase, NPAD // NS)], stripe_v)
    pltpu.sync_copy(stripe_v, degp.at[cid, pl.ds(nbase, NPAD // NS)])


def _deg_stage(dstl, ewl):
    f = pl.kernel(
        _k2_body,
        out_type=jax.ShapeDtypeStruct((NC, NPAD), jnp.float32),
        mesh=_mesh(),
        compiler_params=pltpu.CompilerParams(use_tc_tiling_on_sc=False),
        scratch_types=[
            pltpu.VMEM((1024,), jnp.int32),
            pltpu.VMEM((1024,), jnp.float32),
            pltpu.VMEM((NPAD // NS,), jnp.float32),
            pltpu.VMEM_SHARED((NPAD,), jnp.float32),
            pltpu.SemaphoreType.DMA,
        ],
    )
    return f(dstl, ewl)


def _k3_body(srcl, dstl, ewl, dis, norml, cp,
             sidx0, sidx1, sidx2, didx0, didx1, didx2, ewv0, ewv1, ewv2,
             dsv0, dsv1, dsv2, ddv0, ddv1, ddv2, nv0, nv1, nv2,
             stripe_v, acc_sh,
             sl0, sl1, sl2, sg0, sg1, sg2, ss0, ss1, ss2, sw0, sw1, sw2):
    cid = lax.axis_index("c")
    sid = lax.axis_index("s")
    wid = sid * NC + cid
    nbase = pl.multiple_of(sid * (NPAD // NS), 8)

    SIDX = (sidx0, sidx1, sidx2)
    DIDX = (didx0, didx1, didx2)
    EWV = (ewv0, ewv1, ewv2)
    DSV = (dsv0, dsv1, dsv2)
    DDV = (ddv0, ddv1, ddv2)
    NV = (nv0, nv1, nv2)
    SL = (sl0, sl1, sl2)
    SG = (sg0, sg1, sg2)
    SS = (ss0, ss1, ss2)
    SW = (sw0, sw1, sw2)

    _zero_1d(stripe_v, NPAD // NS)
    pltpu.sync_copy(stripe_v, acc_sh.at[pl.ds(nbase, NPAD // NS)])
    plsc.subcore_barrier()

    ept = EPAD // (NC * NS)
    ebase = wid * ept
    nch = ept // 1024

    def issue_lin(ch, b):
        e0 = ebase + ch * 1024
        pltpu.async_copy(srcl.at[pl.ds(e0, 1024)], SIDX[b], SL[b])
        pltpu.async_copy(dstl.at[pl.ds(e0, 1024)], DIDX[b], SL[b])
        pltpu.async_copy(ewl.at[pl.ds(e0, 1024)], EWV[b], SL[b])

    def wait_lin(b):
        pltpu.make_async_copy(srcl.at[pl.ds(0, 1024)], SIDX[b], SL[b]).wait()
        pltpu.make_async_copy(dstl.at[pl.ds(0, 1024)], DIDX[b], SL[b]).wait()
        pltpu.make_async_copy(ewl.at[pl.ds(0, 1024)], EWV[b], SL[b]).wait()

    def issue_gat(b):
        pltpu.async_copy(dis.at[SIDX[b]], DSV[b], SG[b])
        pltpu.async_copy(dis.at[DIDX[b]], DDV[b], SG[b])

    def wait_gat(b):
        pltpu.make_async_copy(dis.at[SIDX[b]], DSV[b], SG[b]).wait()
        pltpu.make_async_copy(dis.at[DIDX[b]], DDV[b], SG[b]).wait()

    def issue_out(ch, b):
        e0 = ebase + ch * 1024
        pltpu.async_copy(NV[b], norml.at[pl.ds(e0, 1024)], SW[b])
        pltpu.async_copy(NV[b], acc_sh.at[SIDX[b]], SS[b], add=True)

    def wait_out(b):
        pltpu.make_async_copy(NV[b], norml.at[pl.ds(0, 1024)], SW[b]).wait()
        pltpu.make_async_copy(NV[b], acc_sh.at[SIDX[b]], SS[b]).wait()

    def compute(b):
        def f(i, _):
            sl = pl.ds(i * 16, 16)
            NV[b][sl] = DSV[b][sl] * EWV[b][sl] * DDV[b][sl]
            return 0
        lax.fori_loop(0, 64, f, 0)

    def body(ch, b, first=False, has1=True, has2=True):
        nb = (b + 1) % 3
        pb = (b + 2) % 3
        if not first:
            wait_out(pb)
        if has2:
            issue_lin(ch + 2, pb)
        wait_gat(b)
        if has1:
            wait_lin(nb)
            issue_gat(nb)
        compute(b)
        issue_out(ch, b)

    issue_lin(0, 0)
    issue_lin(1, 1)
    wait_lin(0)
    issue_gat(0)
    body(0, 0, first=True)

    def steady(pp, _):
        ch = 3 * pp + 1
        body(ch, 1)
        body(ch + 1, 2)
        body(ch + 2, 0)
        return 0
    lax.fori_loop(0, (nch - 4) // 3, steady, 0)

    body(nch - 3, (nch - 3) % 3)
    body(nch - 2, (nch - 2) % 3, has2=False)
    body(nch - 1, (nch - 1) % 3, has1=False, has2=False)
    wait_out((nch - 1) % 3)

    plsc.subcore_barrier()
    pltpu.sync_copy(acc_sh.at[pl.ds(nbase, NPAD // NS)], stripe_v)
    pltpu.sync_copy(stripe_v, cp.at[cid, pl.ds(nbase, NPAD // NS)])


def _norm_stage(srcl, dstl, ewl, dis):
    f = pl.kernel(
        _k3_body,
        out_type=(
            jax.ShapeDtypeStruct((EPAD,), jnp.float32),
            jax.ShapeDtypeStruct((NC, NPAD), jnp.float32),
        ),
        mesh=_mesh(),
        compiler_params=pltpu.CompilerParams(use_tc_tiling_on_sc=False),
        scratch_types=(
            [pltpu.VMEM((1024,), jnp.int32) for _ in range(6)]
            + [pltpu.VMEM((1024,), jnp.float32) for _ in range(12)]
            + [pltpu.VMEM((NPAD // NS,), jnp.float32)]
            + [pltpu.VMEM_SHARED((NPAD,), jnp.float32)]
            + [pltpu.SemaphoreType.DMA for _ in range(12)]
        ),
    )
    return f(srcl, dstl, ewl, dis)


def _k4_body(srcl, dstl, norml, hflat, agg,
             sidx0, sidx1, sidx2, didx0, didx1, didx2, nv0, nv1, nv2,
             rows0, rows1, rows2, zb, acc_sh,
             sl0, sl1, sl2, sg0, sg1, sg2, ss0, ss1, ss2):
    cid = lax.axis_index("c")
    sid = lax.axis_index("s")
    nbase = pl.multiple_of(sid * (NPAD // NS), 8)

    SIDX = (sidx0, sidx1, sidx2)
    DIDX = (didx0, didx1, didx2)
    NV = (nv0, nv1, nv2)
    ROWS = (rows0, rows1, rows2)
    SL = (sl0, sl1, sl2)
    SG = (sg0, sg1, sg2)
    SS = (ss0, ss1, ss2)

    def zb_zero(i, _):
        zb[i, pl.ds(0, 16)] = jnp.zeros((16,), jnp.float32)
        return 0

    def zero_acc():
        lax.fori_loop(0, 1024, zb_zero, 0)
        for t in range(3):
            pltpu.sync_copy(zb, acc_sh.at[pl.ds(nbase + t * 1024, 1024)])
        pltpu.sync_copy(zb.at[pl.ds(0, 128)],
                        acc_sh.at[pl.ds(nbase + 3072, 128)])

    def dump_acc(q):
        for t in range(3):
            pltpu.sync_copy(acc_sh.at[pl.ds(nbase + t * 1024, 1024)], zb)
            pltpu.sync_copy(zb, agg.at[q, pl.ds(nbase + t * 1024, 1024)])
        pltpu.sync_copy(acc_sh.at[pl.ds(nbase + 3072, 128)],
                        zb.at[pl.ds(0, 128)])
        pltpu.sync_copy(zb.at[pl.ds(0, 128)],
                        agg.at[q, pl.ds(nbase + 3072, 128)])

    edges_per_tile = EPAD // NS
    ebase = sid * edges_per_tile
    nch = edges_per_tile // 1024

    def issue_lin(ch, b):
        e0 = ebase + ch * 1024
        pltpu.async_copy(srcl.at[pl.ds(e0, 1024)], SIDX[b], SL[b])
        pltpu.async_copy(dstl.at[pl.ds(e0, 1024)], DIDX[b], SL[b])
        pltpu.async_copy(norml.at[pl.ds(e0, 1024)], NV[b], SL[b])

    def wait_lin(b):
        pltpu.make_async_copy(srcl.at[pl.ds(0, 1024)], SIDX[b], SL[b]).wait()
        pltpu.make_async_copy(dstl.at[pl.ds(0, 1024)], DIDX[b], SL[b]).wait()
        pltpu.make_async_copy(norml.at[pl.ds(0, 1024)], NV[b], SL[b]).wait()

    def addq(b, qoff):
        def f(i, _):
            sl = pl.ds(i * 16, 16)
            SIDX[b][sl] = SIDX[b][sl] + qoff
            return 0
        lax.fori_loop(0, 64, f, 0)

    def issue_gat(b):
        pltpu.async_copy(hflat.at[SIDX[b]], ROWS[b], SG[b])

    def wait_gat(b):
        pltpu.make_async_copy(hflat.at[SIDX[b]], ROWS[b], SG[b]).wait()

    def issue_sca(b):
        pltpu.async_copy(ROWS[b], acc_sh.at[DIDX[b]], SS[b], add=True)

    def wait_sca(b):
        pltpu.make_async_copy(ROWS[b], acc_sh.at[DIDX[b]], SS[b]).wait()

    def scale(b):
        def f(i, _):
            nvec = NV[b][pl.ds(i * 16, 16)]
            base = i * 16
            for l in range(16):
                ROWS[b][base + l, pl.ds(0, 16)] = (
                    ROWS[b][base + l, pl.ds(0, 16)] * nvec[l])
            return 0
        lax.fori_loop(0, 64, f, 0)

    def body(ch, b, qoff, first=False, has1=True, has2=True):
        nb = (b + 1) % 3
        pb = (b + 2) % 3
        if not first:
            wait_sca(pb)
        if has2:
            issue_lin(ch + 2, pb)
        wait_gat(b)
        if has1:
            wait_lin(nb)
            addq(nb, qoff)
            issue_gat(nb)
        scale(b)
        issue_sca(b)

    def do_pass(p, _):
        q = cid * 2 + p
        qoff = q * NPAD
        zero_acc()
        plsc.subcore_barrier()

        issue_lin(0, 0)
        issue_lin(1, 1)
        wait_lin(0)
        addq(0, qoff)
        issue_gat(0)
        body(0, 0, qoff, first=True)

        def steady(pp, _):
            ch = 3 * pp + 1
            body(ch, 1, qoff)
            body(ch + 1, 2, qoff)
            body(ch + 2, 0, qoff)
            return 0
        lax.fori_loop(0, (nch - 2) // 3, steady, 0)

        body(nch - 1, (nch - 1) % 3, qoff, has1=False, has2=False)
        wait_sca((nch - 1) % 3)

        plsc.subcore_barrier()
        dump_acc(q)
        return 0
    lax.fori_loop(0, 2, do_pass, 0)


def _agg_stage(srcl, dstl, norml, hflat):
    f = pl.kernel(
        _k4_body,
        out_type=jax.ShapeDtypeStruct((4, NPAD, 16), jnp.float32),
        mesh=_mesh(),
        compiler_params=pltpu.CompilerParams(use_tc_tiling_on_sc=False),
        scratch_types=(
            [pltpu.VMEM((1024,), jnp.int32) for _ in range(6)]
            + [pltpu.VMEM((1024,), jnp.float32) for _ in range(3)]
            + [pltpu.VMEM((1024, 16), jnp.float32) for _ in range(4)]
            + [pltpu.VMEM_SHARED((NPAD, 16), jnp.float32)]
            + [pltpu.SemaphoreType.DMA for _ in range(9)]
        ),
    )
    return f(srcl, dstl, norml, hflat)


def _k5_body(agg_ref, h4_ref, dis_ref, cp_ref,
             b1_ref, w2t_ref, b2_ref, out_ref, acc_ref):
    i = pl.program_id(0)
    d = dis_ref[...]
    inv = d * d
    cvec = cp_ref[0] + cp_ref[1] + inv
    b1 = b1_ref[...]
    a = agg_ref[...]
    rowid = jax.lax.broadcasted_iota(jnp.int32, (BN5, 1), 0) + i * BN5
    valid = rowid < NN
    h4 = h4_ref[...]
    hq = [h4[0], h4[1], h4[2], h4[3]]
    parts = []
    for q in range(4):
        gq = jnp.maximum(
            a[q] + hq[q] * inv[:, None] + b1[0, 16 * q:16 * q + 16], 0.0)
        gq = jnp.where(valid, gq, 0.0)
        parts.append(
            jnp.dot(cvec[None, :], gq, preferred_element_type=jnp.float32))
    part = jnp.concatenate(parts, axis=1)

    @pl.when(i == 0)
    def _():
        acc_ref[...] = jnp.zeros_like(acc_ref)

    acc_ref[...] += part

    @pl.when(i == pl.num_programs(0) - 1)
    def _():
        out_ref[...] = (
            jnp.dot(acc_ref[...] * (1.0 / NN), w2t_ref[...],
                    preferred_element_type=jnp.float32)
            + b2_ref[...])


def _finish_stage(agg, h4, dis, cp, b1, w2t, b2):
    return pl.pallas_call(
        _k5_body,
        grid=(NPAD // BN5,),
        in_specs=[
            pl.BlockSpec((4, BN5, 16), lambda i: (0, i, 0)),
            pl.BlockSpec((4, BN5, 16), lambda i: (0, i, 0)),
            pl.BlockSpec((BN5,), lambda i: (i,)),
            pl.BlockSpec((2, BN5), lambda i: (0, i)),
            pl.BlockSpec((1, 64), lambda i: (0, 0)),
            pl.BlockSpec((64, 64), lambda i: (0, 0)),
            pl.BlockSpec((1, 64), lambda i: (0, 0)),
        ],
        out_specs=pl.BlockSpec((1, 64), lambda i: (0, 0)),
        out_shape=jax.ShapeDtypeStruct((1, 64), jnp.float32),
        scratch_shapes=[pltpu.VMEM((1, 64), jnp.float32)],
    )(agg, h4, dis, cp, b1, w2t, b2)


def _band_matrices(conv1_w, conv2_w):
    w1 = conv1_w[0, 0]
    w2 = conv2_w[0, 0]
    i1 = jnp.arange(TT)[:, None]
    s1 = jnp.arange(S1)[None, :]
    d1 = i1 - 4 * s1
    a1f = jnp.zeros((TT, S1), jnp.float32)
    for k in range(KW):
        a1f = a1f + jnp.where(d1 == 2 * k, w1[k], 0.0)
    i2 = jnp.arange(S1)[:, None]
    t2 = jnp.arange(S2)[None, :]
    d2 = i2 - t2
    a2 = jnp.zeros((S1, S2), jnp.float32)
    for k in range(KW):
        a2 = a2 + jnp.where(d2 == k, w2[k], 0.0)
    return a1f, a2


def kernel(node_features, edge_index, edge_attributes, conv1_w, conv1_b,
           conv2_w, conv2_b, W1, b1, W2, b2):
    src = edge_index[0].astype(jnp.int32)
    dst = edge_index[1].astype(jnp.int32)
    ew = edge_attributes.astype(jnp.float32)
    npadlen = EPAD - E0
    pad_idx = jnp.full((npadlen,), NN - 1, jnp.int32)
    srcl = jnp.concatenate([src, pad_idx])
    dstl = jnp.concatenate([dst, pad_idx])
    ewl = jnp.concatenate([ew, jnp.zeros((npadlen,), jnp.float32)])

    a1f, a2 = _band_matrices(conv1_w, conv2_w)
    sb = jnp.stack([conv1_b[0], conv2_b[0]]).reshape(1, 2)

    h4 = _dense_stage(node_features, a1f, a2, W1.T, sb)
    degp = _deg_stage(dstl, ewl)
    dis = _dis_stage(degp)
    norml, cp = _norm_stage(srcl, dstl, ewl, dis)
    agg = _agg_stage(srcl, dstl, norml, h4.reshape(4 * NPAD, 16))
    out = _finish_stage(agg, h4, dis, cp, b1.reshape(1, 64),
                        W2.T, b2.reshape(1, 64))
    return out


# speedup vs baseline: 22.8913x; 1.0161x over previous
import functools
import jax
import jax.numpy as jnp
import numpy as np
from jax import lax
from jax.experimental import pallas as pl
from jax.experimental.pallas import tpu as pltpu
from jax.experimental.pallas import tpu_sc as plsc

NN = 50000
TT = 518
S1 = 114
S2 = 83
KW = 32
E0 = 800000
EPAD = 819200
EROWS = EPAD // 128
NPAD = 51200
NC, NS = 2, 16
BN = 2000
GRID_N = NN // BN
BN5 = 2048

@functools.lru_cache(maxsize=1)
def _mesh():
    return plsc.VectorSubcoreMesh(
        core_axis_name="c", subcore_axis_name="s",
        num_cores=NC, num_subcores=NS)


def _k1_body(x_ref, a1_ref, a2_ref, w1t_ref, sb_ref, h4_ref):
    x = x_ref[...]
    t1 = jnp.maximum(
        jnp.dot(x, a1_ref[...], preferred_element_type=jnp.float32)
        + sb_ref[0, 0], 0.0)
    t2 = jnp.maximum(
        jnp.dot(t1, a2_ref[...], preferred_element_type=jnp.float32)
        + sb_ref[0, 1], 0.0)
    h = jnp.dot(t2, w1t_ref[...], preferred_element_type=jnp.float32)
    h4_ref[...] = jnp.stack(
        [h[:, 0:16], h[:, 16:32], h[:, 32:48], h[:, 48:64]], axis=0)


def _dense_stage(nf, a1f, a2, w1t, sb):
    return pl.pallas_call(
        _k1_body,
        grid=(GRID_N,),
        in_specs=[
            pl.BlockSpec((BN, TT), lambda i: (i, 0)),
            pl.BlockSpec((TT, S1), lambda i: (0, 0)),
            pl.BlockSpec((S1, S2), lambda i: (0, 0)),
            pl.BlockSpec((S2, 64), lambda i: (0, 0)),
            pl.BlockSpec((1, 2), lambda i: (0, 0)),
        ],
        out_specs=pl.BlockSpec((4, BN, 16), lambda i: (0, i, 0)),
        out_shape=jax.ShapeDtypeStruct((4, NPAD, 16), jnp.float32),
    )(nf, a1f, a2, w1t, sb)


def _dis_body(dp_ref, dis_ref):
    p = dp_ref[...]
    dis_ref[...] = lax.rsqrt(1.0 + p[0] + p[1])


def _dis_stage(degp):
    return pl.pallas_call(
        _dis_body,
        grid=(1,),
        in_specs=[pl.BlockSpec((2, NPAD), lambda i: (0, 0))],
        out_specs=pl.BlockSpec((NPAD,), lambda i: (0,)),
        out_shape=jax.ShapeDtypeStruct((NPAD,), jnp.float32),
    )(degp)


def _zero_1d(ref, n):
    def body(i, _):
        ref[pl.ds(i * 16, 16)] = jnp.zeros((16,), jnp.float32)
        return 0
    lax.fori_loop(0, n // 16, body, 0)


def _k2_body(dstl, ewl, degp, idx_v, val_v, stripe_v, acc_sh, sem):
    cid = lax.axis_index("c")
    sid = lax.axis_index("s")
    wid = sid * NC + cid
    nbase = pl.multiple_of(sid * (NPAD // NS), 8)

    _zero_1d(stripe_v, NPAD // NS)
    pltpu.sync_copy(stripe_v, acc_sh.at[pl.ds(nbase, NPAD // NS)])
    plsc.subcore_barrier()

    ept = EPAD // (NC * NS)
    ebase = wid * ept

    def chunk(ch, _):
        e0 = ebase + ch * 1024
        pltpu.sync_copy(dstl.at[pl.ds(e0, 1024)], idx_v)
        pltpu.sync_copy(ewl.at[pl.ds(e0, 1024)], val_v)
        pltpu.sync_copy(val_v, acc_sh.at[idx_v], add=True)
        return 0
    lax.fori_loop(0, ept // 1024, chunk, 0)

    plsc.subcore_barrier()
    pltpu.sync_copy(acc_sh.at[pl.ds(nbase, NPAD // NS)], stripe_v)
    pltpu.sync_copy(stripe_v, degp.at[cid, pl.ds(nbase, NPAD // NS)])


def _deg_stage(dstl, ewl):
    f = pl.kernel(
        _k2_body,
        out_type=jax.ShapeDtypeStruct((NC, NPAD), jnp.float32),
        mesh=_mesh(),
        compiler_params=pltpu.CompilerParams(use_tc_tiling_on_sc=False),
        scratch_types=[
            pltpu.VMEM((1024,), jnp.int32),
            pltpu.VMEM((1024,), jnp.float32),
            pltpu.VMEM((NPAD // NS,), jnp.float32),
            pltpu.VMEM_SHARED((NPAD,), jnp.float32),
            pltpu.SemaphoreType.DMA,
        ],
    )
    return f(dstl, ewl)


def _rsqrt16(x):
    i = plsc.bitcast(x, jnp.int32)
    i = jnp.int32(0x5F3759DF) - lax.shift_right_logical(i, 1)
    y = plsc.bitcast(i, jnp.float32)
    for _ in range(3):
        y = y * (1.5 - 0.5 * x * y * y)
    return y


def _k3_body(srcl, dstl, ewl, degp, norml, cp, dis2,
             sidx0, sidx1, sidx2, osid0, osid1, osid2,
             didx0, didx1, didx2, ewv0, ewv1, ewv2,
             dsv0, dsv1, dsv2, ddv0, ddv1, ddv2, nv0, nv1, nv2,
             stripe_v, p1buf, acc_sh,
             sl0, sl1, sl2, sg0, sg1, sg2, ss0, ss1, ss2, sw0, sw1, sw2):
    cid = lax.axis_index("c")
    sid = lax.axis_index("s")
    wid = sid * NC + cid
    nbase = pl.multiple_of(sid * (NPAD // NS), 8)

    SIDX = (sidx0, sidx1, sidx2)
    OSID = (osid0, osid1, osid2)
    DIDX = (didx0, didx1, didx2)
    EWV = (ewv0, ewv1, ewv2)
    DSV = (dsv0, dsv1, dsv2)
    DDV = (ddv0, ddv1, ddv2)
    NV = (nv0, nv1, nv2)
    SL = (sl0, sl1, sl2)
    SG = (sg0, sg1, sg2)
    SS = (ss0, ss1, ss2)
    SW = (sw0, sw1, sw2)

    pltpu.sync_copy(degp.at[0, pl.ds(nbase, NPAD // NS)], stripe_v)
    pltpu.sync_copy(degp.at[1, pl.ds(nbase, NPAD // NS)], p1buf)

    def disf(i, _):
        sl = pl.ds(i * 16, 16)
        stripe_v[sl] = _rsqrt16(1.0 + stripe_v[sl] + p1buf[sl])
        return 0
    lax.fori_loop(0, (NPAD // NS) // 16, disf, 0)
    doff = pl.multiple_of(cid * NPAD + nbase, 8)
    pltpu.sync_copy(stripe_v, dis2.at[pl.ds(doff, NPAD // NS)])

    _zero_1d(stripe_v, NPAD // NS)
    pltpu.sync_copy(stripe_v, acc_sh.at[pl.ds(nbase, NPAD // NS)])
    plsc.subcore_barrier()

    ept = EPAD // (NC * NS)
    ebase = wid * ept
    nch = ept // 1024
    qoff = cid * NPAD

    def issue_lin(ch, b):
        e0 = ebase + ch * 1024
        pltpu.async_copy(srcl.at[pl.ds(e0, 1024)], SIDX[b], SL[b])
        pltpu.async_copy(dstl.at[pl.ds(e0, 1024)], DIDX[b], SL[b])
        pltpu.async_copy(ewl.at[pl.ds(e0, 1024)], EWV[b], SL[b])

    def wait_lin(b):
        pltpu.make_async_copy(srcl.at[pl.ds(0, 1024)], SIDX[b], SL[b]).wait()
        pltpu.make_async_copy(dstl.at[pl.ds(0, 1024)], DIDX[b], SL[b]).wait()
        pltpu.make_async_copy(ewl.at[pl.ds(0, 1024)], EWV[b], SL[b]).wait()

    def issue_gat(b):
        def f(i, _):
            sl = pl.ds(i * 16, 16)
            OSID[b][sl] = SIDX[b][sl] + qoff
            DIDX[b][sl] = DIDX[b][sl] + qoff
            return 0
        lax.fori_loop(0, 64, f, 0)
        pltpu.async_copy(dis2.at[OSID[b]], DSV[b], SG[b])
        pltpu.async_copy(dis2.at[DIDX[b]], DDV[b], SG[b])

    def wait_gat(b):
        pltpu.make_async_copy(dis2.at[OSID[b]], DSV[b], SG[b]).wait()
        pltpu.make_async_copy(dis2.at[DIDX[b]], DDV[b], SG[b]).wait()

    def issue_out(ch, b):
        e0 = ebase + ch * 1024
        pltpu.async_copy(NV[b], norml.at[pl.ds(e0, 1024)], SW[b])
        pltpu.async_copy(NV[b], acc_sh.at[SIDX[b]], SS[b], add=True)

    def wait_out(b):
        pltpu.make_async_copy(NV[b], norml.at[pl.ds(0, 1024)], SW[b]).wait()
        pltpu.make_async_copy(NV[b], acc_sh.at[SIDX[b]], SS[b]).wait()

    def compute(b):
        def f(i, _):
            sl = pl.ds(i * 16, 16)
            NV[b][sl] = DSV[b][sl] * EWV[b][sl] * DDV[b][sl]
            return 0
        lax.fori_loop(0, 64, f, 0)

    def body(ch, b, first=False, has1=True, has2=True):
        nb = (b + 1) % 3
        pb = (b + 2) % 3
        if not first:
            wait_out(pb)
        if has2:
            issue_lin(ch + 2, pb)
        wait_gat(b)
        if has1:
            wait_lin(nb)
            issue_gat(nb)
        compute(b)
        issue_out(ch, b)

    issue_lin(0, 0)
    issue_lin(1, 1)
    wait_lin(0)
    issue_gat(0)
    body(0, 0, first=True)

    def steady(pp, _):
        ch = 3 * pp + 1
        body(ch, 1)
        body(ch + 1, 2)
        body(ch + 2, 0)
        return 0
    lax.fori_loop(0, (nch - 4) // 3, steady, 0)

    body(nch - 3, (nch - 3) % 3)
    body(nch - 2, (nch - 2) % 3, has2=False)
    body(nch - 1, (nch - 1) % 3, has1=False, has2=False)
    wait_out((nch - 1) % 3)

    plsc.subcore_barrier()
    pltpu.sync_copy(acc_sh.at[pl.ds(nbase, NPAD // NS)], stripe_v)
    pltpu.sync_copy(stripe_v, cp.at[cid, pl.ds(nbase, NPAD // NS)])


def _norm_stage(srcl, dstl, ewl, degp):
    f = pl.kernel(
        _k3_body,
        out_type=(
            jax.ShapeDtypeStruct((EPAD,), jnp.float32),
            jax.ShapeDtypeStruct((NC, NPAD), jnp.float32),
            jax.ShapeDtypeStruct((NC * NPAD,), jnp.float32),
        ),
        mesh=_mesh(),
        compiler_params=pltpu.CompilerParams(
            use_tc_tiling_on_sc=False, needs_layout_passes=False),
        scratch_types=(
            [pltpu.VMEM((1024,), jnp.int32) for _ in range(9)]
            + [pltpu.VMEM((1024,), jnp.float32) for _ in range(12)]
            + [pltpu.VMEM((NPAD // NS,), jnp.float32) for _ in range(2)]
            + [pltpu.VMEM_SHARED((NPAD,), jnp.float32)]
            + [pltpu.SemaphoreType.DMA for _ in range(12)]
        ),
    )
    return f(srcl, dstl, ewl, degp)


def _k4_body(srcl, dstl, norml, hflat, agg,
             sidx0, sidx1, sidx2, didx0, didx1, didx2, nv0, nv1, nv2,
             rows0, rows1, rows2, zb, acc_sh,
             sl0, sl1, sl2, sg0, sg1, sg2, ss0, ss1, ss2):
    cid = lax.axis_index("c")
    sid = lax.axis_index("s")
    nbase = pl.multiple_of(sid * (NPAD // NS), 8)

    SIDX = (sidx0, sidx1, sidx2)
    DIDX = (didx0, didx1, didx2)
    NV = (nv0, nv1, nv2)
    ROWS = (rows0, rows1, rows2)
    SL = (sl0, sl1, sl2)
    SG = (sg0, sg1, sg2)
    SS = (ss0, ss1, ss2)

    def zb_zero(i, _):
        zb[i, pl.ds(0, 16)] = jnp.zeros((16,), jnp.float32)
        return 0

    def zero_acc():
        lax.fori_loop(0, 1024, zb_zero, 0)
        for t in range(3):
            pltpu.sync_copy(zb, acc_sh.at[pl.ds(nbase + t * 1024, 1024)])
        pltpu.sync_copy(zb.at[pl.ds(0, 128)],
                        acc_sh.at[pl.ds(nbase + 3072, 128)])

    def dump_acc(q):
        for t in range(3):
            pltpu.sync_copy(acc_sh.at[pl.ds(nbase + t * 1024, 1024)], zb)
            pltpu.sync_copy(zb, agg.at[q, pl.ds(nbase + t * 1024, 1024)])
        pltpu.sync_copy(acc_sh.at[pl.ds(nbase + 3072, 128)],
                        zb.at[pl.ds(0, 128)])
        pltpu.sync_copy(zb.at[pl.ds(0, 128)],
                        agg.at[q, pl.ds(nbase + 3072, 128)])

    edges_per_tile = EPAD // NS
    ebase = sid * edges_per_tile
    nch = edges_per_tile // 1024

    def issue_lin(ch, b):
        e0 = ebase + ch * 1024
        pltpu.async_copy(srcl.at[pl.ds(e0, 1024)], SIDX[b], SL[b])
        pltpu.async_copy(dstl.at[pl.ds(e0, 1024)], DIDX[b], SL[b])
        pltpu.async_copy(norml.at[pl.ds(e0, 1024)], NV[b], SL[b])

    def wait_lin(b):
        pltpu.make_async_copy(srcl.at[pl.ds(0, 1024)], SIDX[b], SL[b]).wait()
        pltpu.make_async_copy(dstl.at[pl.ds(0, 1024)], DIDX[b], SL[b]).wait()
        pltpu.make_async_copy(norml.at[pl.ds(0, 1024)], NV[b], SL[b]).wait()

    def addq(b, qoff):
        def f(i, _):
            sl = pl.ds(i * 16, 16)
            SIDX[b][sl] = SIDX[b][sl] + qoff
            return 0
        lax.fori_loop(0, 64, f, 0)

    def issue_gat(b):
        pltpu.async_copy(hflat.at[SIDX[b]], ROWS[b], SG[b])

    def wait_gat(b):
        pltpu.make_async_copy(hflat.at[SIDX[b]], ROWS[b], SG[b]).wait()

    def issue_sca(b):
        pltpu.async_copy(ROWS[b], acc_sh.at[DIDX[b]], SS[b], add=True)

    def wait_sca(b):
        pltpu.make_async_copy(ROWS[b], acc_sh.at[DIDX[b]], SS[b]).wait()

    def scale(b):
        def f(i, _):
            nvec = NV[b][pl.ds(i * 16, 16)]
            base = i * 16
            for l in range(16):
                ROWS[b][base + l, pl.ds(0, 16)] = (
                    ROWS[b][base + l, pl.ds(0, 16)] * nvec[l])
            return 0
        lax.fori_loop(0, 64, f, 0)

    def body(ch, b, qoff, first=False, has1=True, has2=True):
        nb = (b + 1) % 3
        pb = (b + 2) % 3
        if not first:
            wait_sca(pb)
        if has2:
            issue_lin(ch + 2, pb)
        wait_gat(b)
        if has1:
            wait_lin(nb)
            addq(nb, qoff)
            issue_gat(nb)
        scale(b)
        issue_sca(b)

    def do_pass(p, _):
        q = cid * 2 + p
        qoff = q * NPAD
        zero_acc()
        plsc.subcore_barrier()

        issue_lin(0, 0)
        issue_lin(1, 1)
        wait_lin(0)
        addq(0, qoff)
        issue_gat(0)
        body(0, 0, qoff, first=True)

        def steady(pp, _):
            ch = 3 * pp + 1
            body(ch, 1, qoff)
            body(ch + 1, 2, qoff)
            body(ch + 2, 0, qoff)
            return 0
        lax.fori_loop(0, (nch - 2) // 3, steady, 0)

        body(nch - 1, (nch - 1) % 3, qoff, has1=False, has2=False)
        wait_sca((nch - 1) % 3)

        plsc.subcore_barrier()
        dump_acc(q)
        return 0
    lax.fori_loop(0, 2, do_pass, 0)


def _agg_stage(srcl, dstl, norml, hflat):
    f = pl.kernel(
        _k4_body,
        out_type=jax.ShapeDtypeStruct((4, NPAD, 16), jnp.float32),
        mesh=_mesh(),
        compiler_params=pltpu.CompilerParams(use_tc_tiling_on_sc=False),
        scratch_types=(
            [pltpu.VMEM((1024,), jnp.int32) for _ in range(6)]
            + [pltpu.VMEM((1024,), jnp.float32) for _ in range(3)]
            + [pltpu.VMEM((1024, 16), jnp.float32) for _ in range(4)]
            + [pltpu.VMEM_SHARED((NPAD, 16), jnp.float32)]
            + [pltpu.SemaphoreType.DMA for _ in range(9)]
        ),
    )
    return f(srcl, dstl, norml, hflat)


def _k5_body(agg_ref, h4_ref, dis_ref, cp_ref,
             b1_ref, w2t_ref, b2_ref, out_ref, acc_ref):
    i = pl.program_id(0)
    d = dis_ref[...]
    inv = d * d
    cvec = cp_ref[0] + cp_ref[1] + inv
    b1 = b1_ref[...]
    a = agg_ref[...]
    rowid = jax.lax.broadcasted_iota(jnp.int32, (BN5, 1), 0) + i * BN5
    valid = rowid < NN
    h4 = h4_ref[...]
    hq = [h4[0], h4[1], h4[2], h4[3]]
    parts = []
    for q in range(4):
        gq = jnp.maximum(
            a[q] + hq[q] * inv[:, None] + b1[0, 16 * q:16 * q + 16], 0.0)
        gq = jnp.where(valid, gq, 0.0)
        parts.append(
            jnp.dot(cvec[None, :], gq, preferred_element_type=jnp.float32))
    part = jnp.concatenate(parts, axis=1)

    @pl.when(i == 0)
    def _():
        acc_ref[...] = jnp.zeros_like(acc_ref)

    acc_ref[...] += part

    @pl.when(i == pl.num_programs(0) - 1)
    def _():
        out_ref[...] = (
            jnp.dot(acc_ref[...] * (1.0 / NN), w2t_ref[...],
                    preferred_element_type=jnp.float32)
            + b2_ref[...])


def _finish_stage(agg, h4, dis, cp, b1, w2t, b2):
    return pl.pallas_call(
        _k5_body,
        grid=(NPAD // BN5,),
        in_specs=[
            pl.BlockSpec((4, BN5, 16), lambda i: (0, i, 0)),
            pl.BlockSpec((4, BN5, 16), lambda i: (0, i, 0)),
            pl.BlockSpec((BN5,), lambda i: (i,)),
            pl.BlockSpec((2, BN5), lambda i: (0, i)),
            pl.BlockSpec((1, 64), lambda i: (0, 0)),
            pl.BlockSpec((64, 64), lambda i: (0, 0)),
            pl.BlockSpec((1, 64), lambda i: (0, 0)),
        ],
        out_specs=pl.BlockSpec((1, 64), lambda i: (0, 0)),
        out_shape=jax.ShapeDtypeStruct((1, 64), jnp.float32),
        scratch_shapes=[pltpu.VMEM((1, 64), jnp.float32)],
    )(agg, h4, dis, cp, b1, w2t, b2)


def _band_matrices(conv1_w, conv2_w):
    w1 = conv1_w[0, 0]
    w2 = conv2_w[0, 0]
    i1 = jnp.arange(TT)[:, None]
    s1 = jnp.arange(S1)[None, :]
    d1 = i1 - 4 * s1
    a1f = jnp.zeros((TT, S1), jnp.float32)
    for k in range(KW):
        a1f = a1f + jnp.where(d1 == 2 * k, w1[k], 0.0)
    i2 = jnp.arange(S1)[:, None]
    t2 = jnp.arange(S2)[None, :]
    d2 = i2 - t2
    a2 = jnp.zeros((S1, S2), jnp.float32)
    for k in range(KW):
        a2 = a2 + jnp.where(d2 == k, w2[k], 0.0)
    return a1f, a2


def kernel(node_features, edge_index, edge_attributes, conv1_w, conv1_b,
           conv2_w, conv2_b, W1, b1, W2, b2):
    src = edge_index[0].astype(jnp.int32)
    dst = edge_index[1].astype(jnp.int32)
    ew = edge_attributes.astype(jnp.float32)
    npadlen = EPAD - E0
    pad_idx = jnp.full((npadlen,), NN - 1, jnp.int32)
    srcl = jnp.concatenate([src, pad_idx])
    dstl = jnp.concatenate([dst, pad_idx])
    ewl = jnp.concatenate([ew, jnp.zeros((npadlen,), jnp.float32)])

    a1f, a2 = _band_matrices(conv1_w, conv2_w)
    sb = jnp.stack([conv1_b[0], conv2_b[0]]).reshape(1, 2)

    h4 = _dense_stage(node_features, a1f, a2, W1.T, sb)
    degp = _deg_stage(dstl, ewl)
    norml, cp, dis = _norm_stage(srcl, dstl, ewl, degp)
    agg = _agg_stage(srcl, dstl, norml, h4.reshape(4 * NPAD, 16))
    out = _finish_stage(agg, h4, dis, cp, b1.reshape(1, 64),
                        W2.T, b2.reshape(1, 64))
    return out
